# Initial kernel scaffold; baseline (speedup 1.0000x reference)
import jax
import jax.numpy as jnp
from jax.experimental import pallas as pl


def kernel(verts, edges, mesh_idx, W0_0, b0_0, W1_0, b1_0, gamma0, beta0, W0_1, b0_1, W1_1, b1_1, gamma1, beta1, fc1_w, fc1_b, style_w, style_b, sem_w, sem_b, func_w, func_b, aes_w, aes_b):
    raise NotImplementedError("write your pallas kernel here")


# speedup vs baseline: 2.0546x; 2.0546x over previous
import functools

import jax
import jax.numpy as jnp
from jax import lax
from jax.experimental import pallas as pl
from jax.experimental.pallas import tpu as pltpu
from jax.experimental.pallas import tpu_sc as plsc

N = 10000
D = 128
M = 32
E = 320000
EPAIR = 2 * E
NW = 32
CHUNK = 128
PER_TILE = 20480
EPAD = PER_TILE * NW
NCHUNK = PER_TILE // CHUNK
NP = 10240
ZROWS = NP // 16

R = 1000
G = N // R

_P = None


def _sc_edge_scatter(x1, dst, src):
    mesh = plsc.VectorSubcoreMesh(core_axis_name="c", subcore_axis_name="s")

    @functools.partial(
        pl.kernel,
        out_type=jax.ShapeDtypeStruct((2 * NP, D), jnp.float32),
        mesh=mesh,
        scratch_types=[
            pltpu.VMEM((CHUNK,), jnp.int32),
            pltpu.VMEM((CHUNK,), jnp.int32),
            pltpu.VMEM((CHUNK, D), jnp.float32),
            pltpu.VMEM((64, D), jnp.float32),
            pltpu.VMEM_SHARED((NP, D), jnp.float32),
            pltpu.SemaphoreType.DMA,
        ],
    )
    def body(x1_hbm, dst_hbm, src_hbm, out_hbm, dstv, srcv, rows, zbuf, agg, sem):
        cid = lax.axis_index("c")
        sid = lax.axis_index("s")
        wid = sid * 2 + cid

        def zrow(r, carry):
            for c8 in range(8):
                zbuf[r, pl.ds(c8 * 16, 16)] = jnp.zeros((16,), jnp.float32)
            return carry

        lax.fori_loop(0, 64, zrow, 0)

        def zcopy(k, carry):
            pltpu.sync_copy(zbuf, agg.at[pl.ds(sid * ZROWS + k * 64, 64)])
            return carry

        lax.fori_loop(0, ZROWS // 64, zcopy, 0)
        plsc.subcore_barrier()

        base0 = wid * PER_TILE

        def step(g, carry):
            base = base0 + g * CHUNK
            pltpu.sync_copy(dst_hbm.at[pl.ds(base, CHUNK)], dstv)
            pltpu.sync_copy(src_hbm.at[pl.ds(base, CHUNK)], srcv)
            pltpu.async_copy(x1_hbm.at[srcv], rows, sem).wait()
            pltpu.sync_copy(rows, agg.at[dstv], add=True)
            return carry

        lax.fori_loop(0, NCHUNK, step, 0)
        plsc.subcore_barrier()
        pltpu.sync_copy(
            agg.at[pl.ds(sid * ZROWS, ZROWS)],
            out_hbm.at[pl.ds(cid * NP + sid * ZROWS, ZROWS)],
        )

    return body(x1, dst, src)


def _mm2(x, w0, b0, w1, b1):
    def body(x_ref, w0_ref, b0_ref, w1_ref, b1_ref, o0_ref, o1_ref):
        xb = x_ref[...]
        o0_ref[...] = jnp.dot(xb, w0_ref[...], precision=_P,
                              preferred_element_type=jnp.float32) + b0_ref[...]
        o1_ref[...] = jnp.dot(xb, w1_ref[...], precision=_P,
                              preferred_element_type=jnp.float32) + b1_ref[...]

    wspec = pl.BlockSpec((D, D), lambda i: (0, 0))
    bspec = pl.BlockSpec((1, D), lambda i: (0, 0))
    rspec = pl.BlockSpec((R, D), lambda i: (i, 0))
    return pl.pallas_call(
        body,
        grid=(G,),
        in_specs=[rspec, wspec, bspec, wspec, bspec],
        out_specs=[rspec, rspec],
        out_shape=[jax.ShapeDtypeStruct((N, D), jnp.float32)] * 2,
    )(x, w0, b0, w1, b1)


def _combine(x0, p):
    def body(x0_ref, p_ref, y_ref, s_ref):
        i = pl.program_id(0)
        y = x0_ref[...] + p_ref[0] + p_ref[1]
        y_ref[...] = y
        st = jnp.concatenate(
            [jnp.sum(y, axis=0, keepdims=True),
             jnp.sum(y * y, axis=0, keepdims=True)], axis=0)

        @pl.when(i == 0)
        def _():
            s_ref[...] = st

        @pl.when(i > 0)
        def _():
            s_ref[...] += st

    rspec = pl.BlockSpec((R, D), lambda i: (i, 0))
    return pl.pallas_call(
        body,
        grid=(G,),
        in_specs=[rspec, pl.BlockSpec((2, R, D), lambda i: (0, i, 0))],
        out_specs=[rspec, pl.BlockSpec((2, D), lambda i: (0, 0))],
        out_shape=[jax.ShapeDtypeStruct((N, D), jnp.float32),
                   jax.ShapeDtypeStruct((2, D), jnp.float32)],
    )(x0, p)


def _bn_relu_mm2(y, s, g, be, w0, b0, w1, b1):
    def body(y_ref, s_ref, g_ref, be_ref, w0_ref, b0_ref, w1_ref, b1_ref,
             o0_ref, o1_ref):
        mu = s_ref[0:1, :] * (1.0 / N)
        ex2 = s_ref[1:2, :] * (1.0 / N)
        inv = lax.rsqrt(ex2 - mu * mu + 1e-5)
        xn = jnp.maximum((y_ref[...] - mu) * (inv * g_ref[...]) + be_ref[...], 0.0)
        o0_ref[...] = jnp.dot(xn, w0_ref[...], precision=_P,
                              preferred_element_type=jnp.float32) + b0_ref[...]
        o1_ref[...] = jnp.dot(xn, w1_ref[...], precision=_P,
                              preferred_element_type=jnp.float32) + b1_ref[...]

    wspec = pl.BlockSpec((D, D), lambda i: (0, 0))
    bspec = pl.BlockSpec((1, D), lambda i: (0, 0))
    rspec = pl.BlockSpec((R, D), lambda i: (i, 0))
    return pl.pallas_call(
        body,
        grid=(G,),
        in_specs=[rspec, pl.BlockSpec((2, D), lambda i: (0, 0)),
                  bspec, bspec, wspec, bspec, wspec, bspec],
        out_specs=[rspec, rspec],
        out_shape=[jax.ShapeDtypeStruct((N, D), jnp.float32)] * 2,
    )(y, s, g, be, w0, b0, w1, b1)


def _pool_heads(y, s, g, be, ids3, fw, fb, hw, hb):
    def body(y_ref, s_ref, g_ref, be_ref, ids_ref, fw_ref, fb_ref, hw_ref,
             hb_ref, o_ref, seg_acc, cnt_acc):
        i = pl.program_id(0)

        @pl.when(i == 0)
        def _():
            seg_acc[...] = jnp.zeros_like(seg_acc)
            cnt_acc[...] = jnp.zeros_like(cnt_acc)

        mu = s_ref[0:1, :] * (1.0 / N)
        ex2 = s_ref[1:2, :] * (1.0 / N)
        inv = lax.rsqrt(ex2 - mu * mu + 1e-5)
        xn = jnp.maximum((y_ref[...] - mu) * (inv * g_ref[...]) + be_ref[...], 0.0)
        ids = ids_ref[0]
        onehot = (jnp.broadcast_to(ids, (M, R))
                  == lax.broadcasted_iota(jnp.int32, (M, R), 0)).astype(jnp.float32)
        seg_acc[...] += jnp.dot(onehot, xn, precision=_P,
                                preferred_element_type=jnp.float32)
        cnt_acc[...] += jnp.broadcast_to(
            jnp.sum(onehot, axis=1, keepdims=True), (M, D))

        @pl.when(i == G - 1)
        def _():
            mean = seg_acc[...] / jnp.maximum(cnt_acc[...], 1.0)
            h = jnp.maximum(
                jnp.dot(mean, fw_ref[...], precision=_P,
                        preferred_element_type=jnp.float32) + fb_ref[...], 0.0)
            o_ref[...] = jnp.dot(h, hw_ref[...], precision=_P,
                                 preferred_element_type=jnp.float32) + hb_ref[...]

    rspec = pl.BlockSpec((R, D), lambda i: (i, 0))
    bspec = pl.BlockSpec((1, D), lambda i: (0, 0))
    wspec = pl.BlockSpec((D, D), lambda i: (0, 0))
    return pl.pallas_call(
        body,
        grid=(G,),
        in_specs=[rspec, pl.BlockSpec((2, D), lambda i: (0, 0)), bspec, bspec,
                  pl.BlockSpec((1, 1, R), lambda i: (i, 0, 0)),
                  wspec, bspec, wspec, bspec],
        out_specs=pl.BlockSpec((M, D), lambda i: (0, 0)),
        out_shape=jax.ShapeDtypeStruct((M, D), jnp.float32),
        scratch_shapes=[pltpu.VMEM((M, D), jnp.float32),
                        pltpu.VMEM((M, D), jnp.float32)],
    )(y, s, g, be, ids3, fw, fb, hw, hb)


def kernel(verts, edges, mesh_idx, W0_0, b0_0, W1_0, b1_0, gamma0, beta0,
           W0_1, b0_1, W1_1, b1_1, gamma1, beta1, fc1_w, fc1_b,
           style_w, style_b, sem_w, sem_b, func_w, func_b, aes_w, aes_b):
    e0 = edges[:, 0].astype(jnp.int32)
    e1 = edges[:, 1].astype(jnp.int32)
    npad = EPAD - EPAIR
    dst = jnp.concatenate([e0, e1, jnp.full((npad,), N, jnp.int32)])
    src = jnp.concatenate([e1, e0, jnp.zeros((npad,), jnp.int32)])

    row = lambda v: v.reshape(1, D)
    ids3 = mesh_idx.astype(jnp.int32).reshape(G, 1, R)
    hw = jnp.pad(jnp.concatenate([style_w, sem_w, func_w, aes_w], axis=1),
                 ((0, 0), (0, D - 14)))
    hb = jnp.pad(jnp.concatenate([style_b, sem_b, func_b, aes_b]).reshape(1, 14),
                 ((0, 0), (0, D - 14)))

    x0, x1 = _mm2(verts, W0_0, row(b0_0), W1_0, row(b1_0))
    p = _sc_edge_scatter(x1, dst, src).reshape(2, NP, D)[:, :N, :]
    y, s = _combine(x0, p)
    x0, x1 = _bn_relu_mm2(y, s, row(gamma0), row(beta0),
                          W0_1, row(b0_1), W1_1, row(b1_1))
    p = _sc_edge_scatter(x1, dst, src).reshape(2, NP, D)[:, :N, :]
    y, s = _combine(x0, p)
    out = _pool_heads(y, s, row(gamma1), row(beta1), ids3,
                      fc1_w, row(fc1_b), hw, hb)
    return (out[:, 0:3], out[:, 3:5], out[:, 5:9], out[:, 9:14])


# speedup vs baseline: 2.5567x; 1.2444x over previous
import functools

import jax
import jax.numpy as jnp
from jax import lax
from jax.experimental import pallas as pl
from jax.experimental.pallas import tpu as pltpu
from jax.experimental.pallas import tpu_sc as plsc

N = 10000
D = 128
M = 32
E = 320000
EPAIR = 2 * E
NW = 32
CHUNK = 128
PER_TILE = 20480
EPAD = PER_TILE * NW
NCHUNK = PER_TILE // CHUNK
NP = 10240
ZROWS = NP // 16

R = 1000
G = N // R

_P = None


NBUF = 2
NI = 8
UNROLL = 8


def _sc_edge_scatter(x1, dst2, src2):
    mesh = plsc.VectorSubcoreMesh(core_axis_name="c", subcore_axis_name="s")

    @functools.partial(
        pl.kernel,
        out_type=jax.ShapeDtypeStruct((2 * NP, D), jnp.float32),
        mesh=mesh,
        scratch_types=[
            [pltpu.VMEM((CHUNK,), jnp.int32)] * NI,
            [pltpu.VMEM((CHUNK,), jnp.int32)] * NI,
            [pltpu.VMEM((CHUNK, D), jnp.float32)] * NBUF,
            pltpu.VMEM((8, D), jnp.float32),
            pltpu.VMEM_SHARED((NP, D), jnp.float32),
            [pltpu.SemaphoreType.DMA] * NI,
            [pltpu.SemaphoreType.DMA] * NBUF,
        ],
    )
    def body(x1_hbm, dst_hbm, src_hbm, out_hbm, dstv, srcv, rbufs, zbuf, agg,
             isems, rsems):
        cid = lax.axis_index("c")
        sid = lax.axis_index("s")
        wid = sid * 2 + cid
        base = wid * NCHUNK

        def idx_issue(slot, h):
            pltpu.async_copy(dst_hbm.at[base + h], dstv[slot], isems[slot])
            pltpu.async_copy(src_hbm.at[base + h], srcv[slot], isems[slot])

        def idx_wait(slot, h):
            pltpu.make_async_copy(dst_hbm.at[base + h], dstv[slot],
                                  isems[slot]).wait()
            pltpu.make_async_copy(src_hbm.at[base + h], srcv[slot],
                                  isems[slot]).wait()

        def gather_issue(slot, islot):
            pltpu.async_copy(x1_hbm.at[srcv[islot]], rbufs[slot], rsems[slot])

        def gather_wait(slot, islot):
            pltpu.make_async_copy(x1_hbm.at[srcv[islot]], rbufs[slot],
                                  rsems[slot]).wait()

        for h in range(NI):
            idx_issue(h, h)

        def zrow(r, carry):
            for c8 in range(8):
                zbuf[r, pl.ds(c8 * 16, 16)] = jnp.zeros((16,), jnp.float32)
            return carry

        lax.fori_loop(0, 8, zrow, 0)

        def zcopy(k, carry):
            pltpu.sync_copy(zbuf, agg.at[pl.ds(sid * ZROWS + k * 8, 8)])
            return carry

        lax.fori_loop(0, ZROWS // 8, zcopy, 0)
        plsc.subcore_barrier()

        for b in range(NBUF - 1):
            idx_wait(b, b)
            gather_issue(b, b)

        def step(t, carry):
            for u in range(UNROLL):
                g = t * UNROLL + u
                nx = g + NBUF - 1
                nu = (u + NBUF - 1) % NBUF
                ni = (u + NBUF - 1) % NI

                @pl.when(nx < NCHUNK)
                def _():
                    idx_wait(ni, nx)
                    gather_issue(nu, ni)

                gather_wait(u % NBUF, u % NI)
                pltpu.sync_copy(rbufs[u % NBUF], agg.at[dstv[u % NI]], add=True)

                @pl.when(g + NI < NCHUNK)
                def _():
                    idx_issue(u % NI, g + NI)
            return carry

        lax.fori_loop(0, NCHUNK // UNROLL, step, 0)
        plsc.subcore_barrier()
        pltpu.sync_copy(
            agg.at[pl.ds(sid * ZROWS, ZROWS)],
            out_hbm.at[pl.ds(cid * NP + sid * ZROWS, ZROWS)],
        )

    return body(x1, dst2, src2)


def _mm2(x, w0, b0, w1, b1):
    def body(x_ref, w0_ref, b0_ref, w1_ref, b1_ref, o0_ref, o1_ref):
        xb = x_ref[...]
        o0_ref[...] = jnp.dot(xb, w0_ref[...], precision=_P,
                              preferred_element_type=jnp.float32) + b0_ref[...]
        o1_ref[...] = jnp.dot(xb, w1_ref[...], precision=_P,
                              preferred_element_type=jnp.float32) + b1_ref[...]

    wspec = pl.BlockSpec((D, D), lambda i: (0, 0))
    bspec = pl.BlockSpec((1, D), lambda i: (0, 0))
    rspec = pl.BlockSpec((R, D), lambda i: (i, 0))
    return pl.pallas_call(
        body,
        grid=(G,),
        in_specs=[rspec, wspec, bspec, wspec, bspec],
        out_specs=[rspec, rspec],
        out_shape=[jax.ShapeDtypeStruct((N, D), jnp.float32)] * 2,
    )(x, w0, b0, w1, b1)


def _combine(x0, p):
    def body(x0_ref, p_ref, y_ref, s_ref):
        i = pl.program_id(0)
        y = x0_ref[...] + p_ref[0] + p_ref[1]
        y_ref[...] = y
        st = jnp.concatenate(
            [jnp.sum(y, axis=0, keepdims=True),
             jnp.sum(y * y, axis=0, keepdims=True)], axis=0)

        @pl.when(i == 0)
        def _():
            s_ref[...] = st

        @pl.when(i > 0)
        def _():
            s_ref[...] += st

    rspec = pl.BlockSpec((R, D), lambda i: (i, 0))
    return pl.pallas_call(
        body,
        grid=(G,),
        in_specs=[rspec, pl.BlockSpec((2, R, D), lambda i: (0, i, 0))],
        out_specs=[rspec, pl.BlockSpec((2, D), lambda i: (0, 0))],
        out_shape=[jax.ShapeDtypeStruct((N, D), jnp.float32),
                   jax.ShapeDtypeStruct((2, D), jnp.float32)],
    )(x0, p)


def _bn_relu_mm2(y, s, g, be, w0, b0, w1, b1):
    def body(y_ref, s_ref, g_ref, be_ref, w0_ref, b0_ref, w1_ref, b1_ref,
             o0_ref, o1_ref):
        mu = s_ref[0:1, :] * (1.0 / N)
        ex2 = s_ref[1:2, :] * (1.0 / N)
        inv = lax.rsqrt(ex2 - mu * mu + 1e-5)
        xn = jnp.maximum((y_ref[...] - mu) * (inv * g_ref[...]) + be_ref[...], 0.0)
        o0_ref[...] = jnp.dot(xn, w0_ref[...], precision=_P,
                              preferred_element_type=jnp.float32) + b0_ref[...]
        o1_ref[...] = jnp.dot(xn, w1_ref[...], precision=_P,
                              preferred_element_type=jnp.float32) + b1_ref[...]

    wspec = pl.BlockSpec((D, D), lambda i: (0, 0))
    bspec = pl.BlockSpec((1, D), lambda i: (0, 0))
    rspec = pl.BlockSpec((R, D), lambda i: (i, 0))
    return pl.pallas_call(
        body,
        grid=(G,),
        in_specs=[rspec, pl.BlockSpec((2, D), lambda i: (0, 0)),
                  bspec, bspec, wspec, bspec, wspec, bspec],
        out_specs=[rspec, rspec],
        out_shape=[jax.ShapeDtypeStruct((N, D), jnp.float32)] * 2,
    )(y, s, g, be, w0, b0, w1, b1)


def _pool_heads(y, s, g, be, ids3, fw, fb, hw, hb):
    def body(y_ref, s_ref, g_ref, be_ref, ids_ref, fw_ref, fb_ref, hw_ref,
             hb_ref, o_ref, seg_acc, cnt_acc):
        i = pl.program_id(0)

        @pl.when(i == 0)
        def _():
            seg_acc[...] = jnp.zeros_like(seg_acc)
            cnt_acc[...] = jnp.zeros_like(cnt_acc)

        mu = s_ref[0:1, :] * (1.0 / N)
        ex2 = s_ref[1:2, :] * (1.0 / N)
        inv = lax.rsqrt(ex2 - mu * mu + 1e-5)
        xn = jnp.maximum((y_ref[...] - mu) * (inv * g_ref[...]) + be_ref[...], 0.0)
        ids = ids_ref[0]
        onehot = (jnp.broadcast_to(ids, (M, R))
                  == lax.broadcasted_iota(jnp.int32, (M, R), 0)).astype(jnp.float32)
        seg_acc[...] += jnp.dot(onehot, xn, precision=_P,
                                preferred_element_type=jnp.float32)
        cnt_acc[...] += jnp.broadcast_to(
            jnp.sum(onehot, axis=1, keepdims=True), (M, D))

        @pl.when(i == G - 1)
        def _():
            mean = seg_acc[...] / jnp.maximum(cnt_acc[...], 1.0)
            h = jnp.maximum(
                jnp.dot(mean, fw_ref[...], precision=_P,
                        preferred_element_type=jnp.float32) + fb_ref[...], 0.0)
            o_ref[...] = jnp.dot(h, hw_ref[...], precision=_P,
                                 preferred_element_type=jnp.float32) + hb_ref[...]

    rspec = pl.BlockSpec((R, D), lambda i: (i, 0))
    bspec = pl.BlockSpec((1, D), lambda i: (0, 0))
    wspec = pl.BlockSpec((D, D), lambda i: (0, 0))
    return pl.pallas_call(
        body,
        grid=(G,),
        in_specs=[rspec, pl.BlockSpec((2, D), lambda i: (0, 0)), bspec, bspec,
                  pl.BlockSpec((1, 1, R), lambda i: (i, 0, 0)),
                  wspec, bspec, wspec, bspec],
        out_specs=pl.BlockSpec((M, D), lambda i: (0, 0)),
        out_shape=jax.ShapeDtypeStruct((M, D), jnp.float32),
        scratch_shapes=[pltpu.VMEM((M, D), jnp.float32),
                        pltpu.VMEM((M, D), jnp.float32)],
    )(y, s, g, be, ids3, fw, fb, hw, hb)


def kernel(verts, edges, mesh_idx, W0_0, b0_0, W1_0, b1_0, gamma0, beta0,
           W0_1, b0_1, W1_1, b1_1, gamma1, beta1, fc1_w, fc1_b,
           style_w, style_b, sem_w, sem_b, func_w, func_b, aes_w, aes_b):
    e0 = edges[:, 0].astype(jnp.int32)
    e1 = edges[:, 1].astype(jnp.int32)
    npad = EPAD - EPAIR
    dst = jnp.concatenate([e0, e1, jnp.full((npad,), N, jnp.int32)]
                          ).reshape(EPAD // CHUNK, CHUNK)
    src = jnp.concatenate([e1, e0, jnp.zeros((npad,), jnp.int32)]
                          ).reshape(EPAD // CHUNK, CHUNK)

    row = lambda v: v.reshape(1, D)
    ids3 = mesh_idx.astype(jnp.int32).reshape(G, 1, R)
    hw = jnp.pad(jnp.concatenate([style_w, sem_w, func_w, aes_w], axis=1),
                 ((0, 0), (0, D - 14)))
    hb = jnp.pad(jnp.concatenate([style_b, sem_b, func_b, aes_b]).reshape(1, 14),
                 ((0, 0), (0, D - 14)))

    x0, x1 = _mm2(verts, W0_0, row(b0_0), W1_0, row(b1_0))
    p = _sc_edge_scatter(x1, dst, src).reshape(2, NP, D)[:, :N, :]
    y, s = _combine(x0, p)
    x0, x1 = _bn_relu_mm2(y, s, row(gamma0), row(beta0),
                          W0_1, row(b0_1), W1_1, row(b1_1))
    p = _sc_edge_scatter(x1, dst, src).reshape(2, NP, D)[:, :N, :]
    y, s = _combine(x0, p)
    out = _pool_heads(y, s, row(gamma1), row(beta1), ids3,
                      fc1_w, row(fc1_b), hw, hb)
    return (out[:, 0:3], out[:, 3:5], out[:, 5:9], out[:, 9:14])


# speedup vs baseline: 9.8769x; 3.8631x over previous
import functools

import jax
import jax.numpy as jnp
from jax import lax
from jax.experimental import pallas as pl
from jax.experimental.pallas import tpu as pltpu
from jax.experimental.pallas import tpu_sc as plsc

N = 10000
D = 128
M = 32
E = 320000
EPAIR = 2 * E
NW = 32
CHUNK = 128
PER_TILE = 20480
EPAD = PER_TILE * NW
NCHUNK = PER_TILE // CHUNK
NP = 10240
ZROWS = NP // 16

R = 1000
G = N // R

_P = None


NBUF = 2
NI = 8
UNROLL = 8


def _sc_edge_scatter(x1, dst2, src2):
    mesh = plsc.VectorSubcoreMesh(core_axis_name="c", subcore_axis_name="s")

    @functools.partial(
        pl.kernel,
        out_type=jax.ShapeDtypeStruct((2 * NP, D), jnp.float32),
        mesh=mesh,
        scratch_types=[
            [pltpu.VMEM((CHUNK,), jnp.int32)] * NI,
            [pltpu.VMEM((CHUNK,), jnp.int32)] * NI,
            [pltpu.VMEM((CHUNK, D), jnp.float32)] * NBUF,
            pltpu.VMEM((8, D), jnp.float32),
            pltpu.VMEM_SHARED((NP, D), jnp.float32),
            [pltpu.SemaphoreType.DMA] * NI,
            [pltpu.SemaphoreType.DMA] * NBUF,
        ],
    )
    def body(x1_hbm, dst_hbm, src_hbm, out_hbm, dstv, srcv, rbufs, zbuf, agg,
             isems, rsems):
        cid = lax.axis_index("c")
        sid = lax.axis_index("s")
        wid = sid * 2 + cid
        base = wid * NCHUNK

        def idx_issue(slot, h):
            pltpu.async_copy(dst_hbm.at[base + h], dstv[slot], isems[slot])
            pltpu.async_copy(src_hbm.at[base + h], srcv[slot], isems[slot])

        def idx_wait(slot, h):
            pltpu.make_async_copy(dst_hbm.at[base + h], dstv[slot],
                                  isems[slot]).wait()
            pltpu.make_async_copy(src_hbm.at[base + h], srcv[slot],
                                  isems[slot]).wait()

        def gather_issue(slot, islot):
            pltpu.async_copy(x1_hbm.at[srcv[islot]], rbufs[slot], rsems[slot])

        def gather_wait(slot, islot):
            pltpu.make_async_copy(x1_hbm.at[srcv[islot]], rbufs[slot],
                                  rsems[slot]).wait()

        for h in range(NI):
            idx_issue(h, h)

        def zrow(r, carry):
            for c8 in range(8):
                zbuf[r, pl.ds(c8 * 16, 16)] = jnp.zeros((16,), jnp.float32)
            return carry

        lax.fori_loop(0, 8, zrow, 0)

        def zcopy(k, carry):
            pltpu.sync_copy(zbuf, agg.at[pl.ds(sid * ZROWS + k * 8, 8)])
            return carry

        lax.fori_loop(0, ZROWS // 8, zcopy, 0)
        plsc.subcore_barrier()

        for b in range(NBUF - 1):
            idx_wait(b, b)
            gather_issue(b, b)

        def step(t, carry):
            for u in range(UNROLL):
                g = t * UNROLL + u
                nx = g + NBUF - 1
                nu = (u + NBUF - 1) % NBUF
                ni = (u + NBUF - 1) % NI

                @pl.when(nx < NCHUNK)
                def _():
                    idx_wait(ni, nx)
                    gather_issue(nu, ni)

                gather_wait(u % NBUF, u % NI)
                pltpu.sync_copy(rbufs[u % NBUF], agg.at[dstv[u % NI]], add=True)

                @pl.when(g + NI < NCHUNK)
                def _():
                    idx_issue(u % NI, g + NI)
            return carry

        lax.fori_loop(0, NCHUNK // UNROLL, step, 0)
        plsc.subcore_barrier()
        pltpu.sync_copy(
            agg.at[pl.ds(sid * ZROWS, ZROWS)],
            out_hbm.at[pl.ds(cid * NP + sid * ZROWS, ZROWS)],
        )

    return body(x1, dst2, src2)


def _mm2(x, w0, b0, w1, b1):
    def body(x_ref, w0_ref, b0_ref, w1_ref, b1_ref, o0_ref, o1_ref):
        xb = x_ref[...]
        o0_ref[...] = jnp.dot(xb, w0_ref[...], precision=_P,
                              preferred_element_type=jnp.float32) + b0_ref[...]
        o1_ref[...] = jnp.dot(xb, w1_ref[...], precision=_P,
                              preferred_element_type=jnp.float32) + b1_ref[...]

    wspec = pl.BlockSpec((D, D), lambda i: (0, 0))
    bspec = pl.BlockSpec((1, D), lambda i: (0, 0))
    rspec = pl.BlockSpec((R, D), lambda i: (i, 0))
    return pl.pallas_call(
        body,
        grid=(G,),
        in_specs=[rspec, wspec, bspec, wspec, bspec],
        out_specs=[rspec, rspec],
        out_shape=[jax.ShapeDtypeStruct((N, D), jnp.float32)] * 2,
    )(x, w0, b0, w1, b1)


def _combine(x0, p):
    def body(x0_ref, p_ref, y_ref, s_ref):
        i = pl.program_id(0)
        y = x0_ref[...] + p_ref[0] + p_ref[1]
        y_ref[...] = y
        st = jnp.concatenate(
            [jnp.sum(y, axis=0, keepdims=True),
             jnp.sum(y * y, axis=0, keepdims=True)], axis=0)

        @pl.when(i == 0)
        def _():
            s_ref[...] = st

        @pl.when(i > 0)
        def _():
            s_ref[...] += st

    rspec = pl.BlockSpec((R, D), lambda i: (i, 0))
    return pl.pallas_call(
        body,
        grid=(G,),
        in_specs=[rspec, pl.BlockSpec((2, R, D), lambda i: (0, i, 0))],
        out_specs=[rspec, pl.BlockSpec((2, D), lambda i: (0, 0))],
        out_shape=[jax.ShapeDtypeStruct((N, D), jnp.float32),
                   jax.ShapeDtypeStruct((2, D), jnp.float32)],
    )(x0, p)


def _bn_relu_mm2(y, s, g, be, w0, b0, w1, b1):
    def body(y_ref, s_ref, g_ref, be_ref, w0_ref, b0_ref, w1_ref, b1_ref,
             o0_ref, o1_ref):
        mu = s_ref[0:1, :] * (1.0 / N)
        ex2 = s_ref[1:2, :] * (1.0 / N)
        inv = lax.rsqrt(ex2 - mu * mu + 1e-5)
        xn = jnp.maximum((y_ref[...] - mu) * (inv * g_ref[...]) + be_ref[...], 0.0)
        o0_ref[...] = jnp.dot(xn, w0_ref[...], precision=_P,
                              preferred_element_type=jnp.float32) + b0_ref[...]
        o1_ref[...] = jnp.dot(xn, w1_ref[...], precision=_P,
                              preferred_element_type=jnp.float32) + b1_ref[...]

    wspec = pl.BlockSpec((D, D), lambda i: (0, 0))
    bspec = pl.BlockSpec((1, D), lambda i: (0, 0))
    rspec = pl.BlockSpec((R, D), lambda i: (i, 0))
    return pl.pallas_call(
        body,
        grid=(G,),
        in_specs=[rspec, pl.BlockSpec((2, D), lambda i: (0, 0)),
                  bspec, bspec, wspec, bspec, wspec, bspec],
        out_specs=[rspec, rspec],
        out_shape=[jax.ShapeDtypeStruct((N, D), jnp.float32)] * 2,
    )(y, s, g, be, w0, b0, w1, b1)


def _pool_heads(y, s, g, be, ids3, fw, fb, hw, hb):
    def body(y_ref, s_ref, g_ref, be_ref, ids_ref, fw_ref, fb_ref, hw_ref,
             hb_ref, o_ref, seg_acc, cnt_acc):
        i = pl.program_id(0)

        @pl.when(i == 0)
        def _():
            seg_acc[...] = jnp.zeros_like(seg_acc)
            cnt_acc[...] = jnp.zeros_like(cnt_acc)

        mu = s_ref[0:1, :] * (1.0 / N)
        ex2 = s_ref[1:2, :] * (1.0 / N)
        inv = lax.rsqrt(ex2 - mu * mu + 1e-5)
        xn = jnp.maximum((y_ref[...] - mu) * (inv * g_ref[...]) + be_ref[...], 0.0)
        ids = ids_ref[0]
        onehot = (jnp.broadcast_to(ids, (M, R))
                  == lax.broadcasted_iota(jnp.int32, (M, R), 0)).astype(jnp.float32)
        seg_acc[...] += jnp.dot(onehot, xn, precision=_P,
                                preferred_element_type=jnp.float32)
        cnt_acc[...] += jnp.broadcast_to(
            jnp.sum(onehot, axis=1, keepdims=True), (M, D))

        @pl.when(i == G - 1)
        def _():
            mean = seg_acc[...] / jnp.maximum(cnt_acc[...], 1.0)
            h = jnp.maximum(
                jnp.dot(mean, fw_ref[...], precision=_P,
                        preferred_element_type=jnp.float32) + fb_ref[...], 0.0)
            o_ref[...] = jnp.dot(h, hw_ref[...], precision=_P,
                                 preferred_element_type=jnp.float32) + hb_ref[...]

    rspec = pl.BlockSpec((R, D), lambda i: (i, 0))
    bspec = pl.BlockSpec((1, D), lambda i: (0, 0))
    wspec = pl.BlockSpec((D, D), lambda i: (0, 0))
    return pl.pallas_call(
        body,
        grid=(G,),
        in_specs=[rspec, pl.BlockSpec((2, D), lambda i: (0, 0)), bspec, bspec,
                  pl.BlockSpec((1, 1, R), lambda i: (i, 0, 0)),
                  wspec, bspec, wspec, bspec],
        out_specs=pl.BlockSpec((M, D), lambda i: (0, 0)),
        out_shape=jax.ShapeDtypeStruct((M, D), jnp.float32),
        scratch_shapes=[pltpu.VMEM((M, D), jnp.float32),
                        pltpu.VMEM((M, D), jnp.float32)],
    )(y, s, g, be, ids3, fw, fb, hw, hb)


def kernel(verts, edges, mesh_idx, W0_0, b0_0, W1_0, b1_0, gamma0, beta0,
           W0_1, b0_1, W1_1, b1_1, gamma1, beta1, fc1_w, fc1_b,
           style_w, style_b, sem_w, sem_b, func_w, func_b, aes_w, aes_b):
    e0 = edges[:, 0].astype(jnp.int32)
    e1 = edges[:, 1].astype(jnp.int32)
    npad = EPAD - EPAIR
    ar = jnp.arange(npad, dtype=jnp.int32)
    dst = jnp.concatenate([e0, e1, N + ar % (NP - N)]
                          ).reshape(EPAD // CHUNK, CHUNK)
    src = jnp.concatenate([e1, e0, ar % N]
                          ).reshape(EPAD // CHUNK, CHUNK)

    row = lambda v: v.reshape(1, D)
    ids3 = mesh_idx.astype(jnp.int32).reshape(G, 1, R)
    hw = jnp.pad(jnp.concatenate([style_w, sem_w, func_w, aes_w], axis=1),
                 ((0, 0), (0, D - 14)))
    hb = jnp.pad(jnp.concatenate([style_b, sem_b, func_b, aes_b]).reshape(1, 14),
                 ((0, 0), (0, D - 14)))

    x0, x1 = _mm2(verts, W0_0, row(b0_0), W1_0, row(b1_0))
    p = _sc_edge_scatter(x1, dst, src).reshape(2, NP, D)[:, :N, :]
    y, s = _combine(x0, p)
    x0, x1 = _bn_relu_mm2(y, s, row(gamma0), row(beta0),
                          W0_1, row(b0_1), W1_1, row(b1_1))
    p = _sc_edge_scatter(x1, dst, src).reshape(2, NP, D)[:, :N, :]
    y, s = _combine(x0, p)
    out = _pool_heads(y, s, row(gamma1), row(beta1), ids3,
                      fc1_w, row(fc1_b), hw, hb)
    return (out[:, 0:3], out[:, 3:5], out[:, 5:9], out[:, 9:14])


# speedup vs baseline: 10.1936x; 1.0321x over previous
import functools

import jax
import jax.numpy as jnp
from jax import lax
from jax.experimental import pallas as pl
from jax.experimental.pallas import tpu as pltpu
from jax.experimental.pallas import tpu_sc as plsc

N = 10000
D = 128
M = 32
E = 320000
EPAIR = 2 * E
NW = 32
CHUNK = 128
PER_TILE = 20480
EPAD = PER_TILE * NW
NCHUNK = PER_TILE // CHUNK
NP = 10240
ZROWS = NP // 16

R = 1000
G = N // R

_P = None


NBUF = 2
NI = 8
UNROLL = 8


def _sc_edge_scatter(x1, dst2, src2):
    mesh = plsc.VectorSubcoreMesh(core_axis_name="c", subcore_axis_name="s")

    @functools.partial(
        pl.kernel,
        out_type=jax.ShapeDtypeStruct((2 * NP, D), jnp.float32),
        mesh=mesh,
        scratch_types=[
            [pltpu.VMEM((CHUNK,), jnp.int32)] * NI,
            [pltpu.VMEM((CHUNK,), jnp.int32)] * NI,
            [pltpu.VMEM((CHUNK, D), jnp.float32)] * NBUF,
            pltpu.VMEM((8, D), jnp.float32),
            pltpu.VMEM_SHARED((NP, D), jnp.float32),
            [pltpu.SemaphoreType.DMA] * NI,
            [pltpu.SemaphoreType.DMA] * NBUF,
            [pltpu.SemaphoreType.DMA] * NBUF,
        ],
    )
    def body(x1_hbm, dst_hbm, src_hbm, out_hbm, dstv, srcv, rbufs, zbuf, agg,
             isems, rsems, ssems):
        cid = lax.axis_index("c")
        sid = lax.axis_index("s")
        wid = sid * 2 + cid
        base = wid * NCHUNK

        def idx_issue(slot, h):
            pltpu.async_copy(dst_hbm.at[base + h], dstv[slot], isems[slot])
            pltpu.async_copy(src_hbm.at[base + h], srcv[slot], isems[slot])

        def idx_wait(slot, h):
            pltpu.make_async_copy(dst_hbm.at[base + h], dstv[slot],
                                  isems[slot]).wait()
            pltpu.make_async_copy(src_hbm.at[base + h], srcv[slot],
                                  isems[slot]).wait()

        def gather_issue(slot, islot):
            pltpu.async_copy(x1_hbm.at[srcv[islot]], rbufs[slot], rsems[slot])

        def gather_wait(slot, islot):
            pltpu.make_async_copy(x1_hbm.at[srcv[islot]], rbufs[slot],
                                  rsems[slot]).wait()

        for h in range(NI):
            idx_issue(h, h)

        def zrow(r, carry):
            for c8 in range(8):
                zbuf[r, pl.ds(c8 * 16, 16)] = jnp.zeros((16,), jnp.float32)
            return carry

        lax.fori_loop(0, 8, zrow, 0)

        def zcopy(k, carry):
            pltpu.sync_copy(zbuf, agg.at[pl.ds(sid * ZROWS + k * 8, 8)])
            return carry

        lax.fori_loop(0, ZROWS // 8, zcopy, 0)
        plsc.subcore_barrier()

        def scatter_issue(slot, islot):
            pltpu.async_copy(rbufs[slot], agg.at[dstv[islot]], ssems[slot],
                             add=True)

        def scatter_wait(slot, islot):
            pltpu.make_async_copy(rbufs[slot], agg.at[dstv[islot]],
                                  ssems[slot]).wait()

        idx_wait(0, 0)
        gather_issue(0, 0)

        def step(t, carry):
            for u in range(UNROLL):
                g = t * UNROLL + u
                nu = (u + 1) % NBUF
                ni = (u + 1) % NI

                @pl.when(g >= 1)
                def _():
                    scatter_wait(nu, ni)

                @pl.when(g + 1 < NCHUNK)
                def _():
                    idx_wait(ni, g + 1)
                    gather_issue(nu, ni)

                gather_wait(u % NBUF, u)
                scatter_issue(u % NBUF, u)

                @pl.when((g >= 1) & (g + NI - 1 < NCHUNK))
                def _():
                    idx_issue((u + NI - 1) % NI, g + NI - 1)
            return carry

        lax.fori_loop(0, NCHUNK // UNROLL, step, 0)
        scatter_wait((NCHUNK - 1) % NBUF, (NCHUNK - 1) % NI)
        plsc.subcore_barrier()
        pltpu.sync_copy(
            agg.at[pl.ds(sid * ZROWS, ZROWS)],
            out_hbm.at[pl.ds(cid * NP + sid * ZROWS, ZROWS)],
        )

    return body(x1, dst2, src2)


def _mm2(x, w0, b0, w1, b1):
    def body(x_ref, w0_ref, b0_ref, w1_ref, b1_ref, o0_ref, o1_ref):
        xb = x_ref[...]
        o0_ref[...] = jnp.dot(xb, w0_ref[...], precision=_P,
                              preferred_element_type=jnp.float32) + b0_ref[...]
        o1_ref[...] = jnp.dot(xb, w1_ref[...], precision=_P,
                              preferred_element_type=jnp.float32) + b1_ref[...]

    wspec = pl.BlockSpec((D, D), lambda i: (0, 0))
    bspec = pl.BlockSpec((1, D), lambda i: (0, 0))
    rspec = pl.BlockSpec((R, D), lambda i: (i, 0))
    return pl.pallas_call(
        body,
        grid=(G,),
        in_specs=[rspec, wspec, bspec, wspec, bspec],
        out_specs=[rspec, rspec],
        out_shape=[jax.ShapeDtypeStruct((N, D), jnp.float32)] * 2,
    )(x, w0, b0, w1, b1)


def _combine(x0, p):
    def body(x0_ref, p_ref, y_ref, s_ref):
        i = pl.program_id(0)
        y = x0_ref[...] + p_ref[0] + p_ref[1]
        y_ref[...] = y
        st = jnp.concatenate(
            [jnp.sum(y, axis=0, keepdims=True),
             jnp.sum(y * y, axis=0, keepdims=True)], axis=0)

        @pl.when(i == 0)
        def _():
            s_ref[...] = st

        @pl.when(i > 0)
        def _():
            s_ref[...] += st

    rspec = pl.BlockSpec((R, D), lambda i: (i, 0))
    return pl.pallas_call(
        body,
        grid=(G,),
        in_specs=[rspec, pl.BlockSpec((2, R, D), lambda i: (0, i, 0))],
        out_specs=[rspec, pl.BlockSpec((2, D), lambda i: (0, 0))],
        out_shape=[jax.ShapeDtypeStruct((N, D), jnp.float32),
                   jax.ShapeDtypeStruct((2, D), jnp.float32)],
    )(x0, p)


def _bn_relu_mm2(y, s, g, be, w0, b0, w1, b1):
    def body(y_ref, s_ref, g_ref, be_ref, w0_ref, b0_ref, w1_ref, b1_ref,
             o0_ref, o1_ref):
        mu = s_ref[0:1, :] * (1.0 / N)
        ex2 = s_ref[1:2, :] * (1.0 / N)
        inv = lax.rsqrt(ex2 - mu * mu + 1e-5)
        xn = jnp.maximum((y_ref[...] - mu) * (inv * g_ref[...]) + be_ref[...], 0.0)
        o0_ref[...] = jnp.dot(xn, w0_ref[...], precision=_P,
                              preferred_element_type=jnp.float32) + b0_ref[...]
        o1_ref[...] = jnp.dot(xn, w1_ref[...], precision=_P,
                              preferred_element_type=jnp.float32) + b1_ref[...]

    wspec = pl.BlockSpec((D, D), lambda i: (0, 0))
    bspec = pl.BlockSpec((1, D), lambda i: (0, 0))
    rspec = pl.BlockSpec((R, D), lambda i: (i, 0))
    return pl.pallas_call(
        body,
        grid=(G,),
        in_specs=[rspec, pl.BlockSpec((2, D), lambda i: (0, 0)),
                  bspec, bspec, wspec, bspec, wspec, bspec],
        out_specs=[rspec, rspec],
        out_shape=[jax.ShapeDtypeStruct((N, D), jnp.float32)] * 2,
    )(y, s, g, be, w0, b0, w1, b1)


def _pool_heads(y, s, g, be, ids3, fw, fb, hw, hb):
    def body(y_ref, s_ref, g_ref, be_ref, ids_ref, fw_ref, fb_ref, hw_ref,
             hb_ref, o_ref, seg_acc, cnt_acc):
        i = pl.program_id(0)

        @pl.when(i == 0)
        def _():
            seg_acc[...] = jnp.zeros_like(seg_acc)
            cnt_acc[...] = jnp.zeros_like(cnt_acc)

        mu = s_ref[0:1, :] * (1.0 / N)
        ex2 = s_ref[1:2, :] * (1.0 / N)
        inv = lax.rsqrt(ex2 - mu * mu + 1e-5)
        xn = jnp.maximum((y_ref[...] - mu) * (inv * g_ref[...]) + be_ref[...], 0.0)
        ids = ids_ref[0]
        onehot = (jnp.broadcast_to(ids, (M, R))
                  == lax.broadcasted_iota(jnp.int32, (M, R), 0)).astype(jnp.float32)
        seg_acc[...] += jnp.dot(onehot, xn, precision=_P,
                                preferred_element_type=jnp.float32)
        cnt_acc[...] += jnp.broadcast_to(
            jnp.sum(onehot, axis=1, keepdims=True), (M, D))

        @pl.when(i == G - 1)
        def _():
            mean = seg_acc[...] / jnp.maximum(cnt_acc[...], 1.0)
            h = jnp.maximum(
                jnp.dot(mean, fw_ref[...], precision=_P,
                        preferred_element_type=jnp.float32) + fb_ref[...], 0.0)
            o_ref[...] = jnp.dot(h, hw_ref[...], precision=_P,
                                 preferred_element_type=jnp.float32) + hb_ref[...]

    rspec = pl.BlockSpec((R, D), lambda i: (i, 0))
    bspec = pl.BlockSpec((1, D), lambda i: (0, 0))
    wspec = pl.BlockSpec((D, D), lambda i: (0, 0))
    return pl.pallas_call(
        body,
        grid=(G,),
        in_specs=[rspec, pl.BlockSpec((2, D), lambda i: (0, 0)), bspec, bspec,
                  pl.BlockSpec((1, 1, R), lambda i: (i, 0, 0)),
                  wspec, bspec, wspec, bspec],
        out_specs=pl.BlockSpec((M, D), lambda i: (0, 0)),
        out_shape=jax.ShapeDtypeStruct((M, D), jnp.float32),
        scratch_shapes=[pltpu.VMEM((M, D), jnp.float32),
                        pltpu.VMEM((M, D), jnp.float32)],
    )(y, s, g, be, ids3, fw, fb, hw, hb)


def kernel(verts, edges, mesh_idx, W0_0, b0_0, W1_0, b1_0, gamma0, beta0,
           W0_1, b0_1, W1_1, b1_1, gamma1, beta1, fc1_w, fc1_b,
           style_w, style_b, sem_w, sem_b, func_w, func_b, aes_w, aes_b):
    e0 = edges[:, 0].astype(jnp.int32)
    e1 = edges[:, 1].astype(jnp.int32)
    npad = EPAD - EPAIR
    ar = jnp.arange(npad, dtype=jnp.int32)
    dst = jnp.concatenate([e0, e1, N + ar % (NP - N)]
                          ).reshape(EPAD // CHUNK, CHUNK)
    src = jnp.concatenate([e1, e0, ar % N]
                          ).reshape(EPAD // CHUNK, CHUNK)

    row = lambda v: v.reshape(1, D)
    ids3 = mesh_idx.astype(jnp.int32).reshape(G, 1, R)
    hw = jnp.pad(jnp.concatenate([style_w, sem_w, func_w, aes_w], axis=1),
                 ((0, 0), (0, D - 14)))
    hb = jnp.pad(jnp.concatenate([style_b, sem_b, func_b, aes_b]).reshape(1, 14),
                 ((0, 0), (0, D - 14)))

    x0, x1 = _mm2(verts, W0_0, row(b0_0), W1_0, row(b1_0))
    p = _sc_edge_scatter(x1, dst, src).reshape(2, NP, D)
    y, s = _combine(x0, p)
    x0, x1 = _bn_relu_mm2(y, s, row(gamma0), row(beta0),
                          W0_1, row(b0_1), W1_1, row(b1_1))
    p = _sc_edge_scatter(x1, dst, src).reshape(2, NP, D)
    y, s = _combine(x0, p)
    out = _pool_heads(y, s, row(gamma1), row(beta1), ids3,
                      fc1_w, row(fc1_b), hw, hb)
    return (out[:, 0:3], out[:, 3:5], out[:, 5:9], out[:, 9:14])


# speedup vs baseline: 11.3325x; 1.1117x over previous
import functools

import jax
import jax.numpy as jnp
from jax import lax
from jax.experimental import pallas as pl
from jax.experimental.pallas import tpu as pltpu
from jax.experimental.pallas import tpu_sc as plsc

N = 10000
D = 128
M = 32
E = 320000
EPAIR = 2 * E
NW = 32
CHUNK = 64
PER_TILE = 20480
EPAD = PER_TILE * NW
NCHUNK = PER_TILE // CHUNK
NP = 10240
ZROWS = NP // 16

R = 1000
G = N // R

_P = None


NBUF = 4
NI = 8
UNROLL = 8


def _sc_edge_scatter(x1, dst2, src2):
    mesh = plsc.VectorSubcoreMesh(core_axis_name="c", subcore_axis_name="s")

    @functools.partial(
        pl.kernel,
        out_type=jax.ShapeDtypeStruct((2 * NP, D), jnp.float32),
        mesh=mesh,
        scratch_types=[
            [pltpu.VMEM((CHUNK,), jnp.int32)] * NI,
            [pltpu.VMEM((CHUNK,), jnp.int32)] * NI,
            [pltpu.VMEM((CHUNK, D), jnp.float32)] * NBUF,
            pltpu.VMEM((8, D), jnp.float32),
            pltpu.VMEM_SHARED((NP, D), jnp.float32),
            [pltpu.SemaphoreType.DMA] * NI,
            [pltpu.SemaphoreType.DMA] * NBUF,
            [pltpu.SemaphoreType.DMA] * NBUF,
        ],
    )
    def body(x1_hbm, dst_hbm, src_hbm, out_hbm, dstv, srcv, rbufs, zbuf, agg,
             isems, rsems, ssems):
        cid = lax.axis_index("c")
        sid = lax.axis_index("s")
        wid = sid * 2 + cid
        base = wid * NCHUNK

        def idx_issue(slot, h):
            pltpu.async_copy(dst_hbm.at[base + h], dstv[slot], isems[slot])
            pltpu.async_copy(src_hbm.at[base + h], srcv[slot], isems[slot])

        def idx_wait(slot, h):
            pltpu.make_async_copy(dst_hbm.at[base + h], dstv[slot],
                                  isems[slot]).wait()
            pltpu.make_async_copy(src_hbm.at[base + h], srcv[slot],
                                  isems[slot]).wait()

        def gather_issue(slot, islot):
            pltpu.async_copy(x1_hbm.at[srcv[islot]], rbufs[slot], rsems[slot])

        def gather_wait(slot, islot):
            pltpu.make_async_copy(x1_hbm.at[srcv[islot]], rbufs[slot],
                                  rsems[slot]).wait()

        for h in range(NI):
            idx_issue(h, h)

        def zrow(r, carry):
            for c8 in range(8):
                zbuf[r, pl.ds(c8 * 16, 16)] = jnp.zeros((16,), jnp.float32)
            return carry

        lax.fori_loop(0, 8, zrow, 0)

        def zcopy(k, carry):
            pltpu.sync_copy(zbuf, agg.at[pl.ds(sid * ZROWS + k * 8, 8)])
            return carry

        lax.fori_loop(0, ZROWS // 8, zcopy, 0)
        plsc.subcore_barrier()

        def scatter_issue(slot, islot):
            pltpu.async_copy(rbufs[slot], agg.at[dstv[islot]], ssems[slot],
                             add=True)

        def scatter_wait(slot, islot):
            pltpu.make_async_copy(rbufs[slot], agg.at[dstv[islot]],
                                  ssems[slot]).wait()

        for h in range(NBUF - 1):
            idx_wait(h, h)
            gather_issue(h, h)

        def step(t, carry):
            for u in range(UNROLL):
                g = t * UNROLL + u
                nx = g + NBUF - 1
                ns = (u + NBUF - 1) % NBUF
                ni = (u + NBUF - 1) % NI

                @pl.when(g >= 1)
                def _():
                    scatter_wait(ns, ni)

                @pl.when(nx < NCHUNK)
                def _():
                    idx_wait(ni, nx)
                    gather_issue(ns, ni)

                gather_wait(u % NBUF, u)
                scatter_issue(u % NBUF, u)

                @pl.when((g >= 1) & (g + NI - 1 < NCHUNK))
                def _():
                    idx_issue((u + NI - 1) % NI, g + NI - 1)
            return carry

        lax.fori_loop(0, NCHUNK // UNROLL, step, 0)
        scatter_wait((NCHUNK - 1) % NBUF, (NCHUNK - 1) % NI)
        plsc.subcore_barrier()
        pltpu.sync_copy(
            agg.at[pl.ds(sid * ZROWS, ZROWS)],
            out_hbm.at[pl.ds(cid * NP + sid * ZROWS, ZROWS)],
        )

    return body(x1, dst2, src2)


def _mm2(x, w0, b0, w1, b1):
    def body(x_ref, w0_ref, b0_ref, w1_ref, b1_ref, o0_ref, o1_ref):
        xb = x_ref[...]
        o0_ref[...] = jnp.dot(xb, w0_ref[...], precision=_P,
                              preferred_element_type=jnp.float32) + b0_ref[...]
        o1_ref[...] = jnp.dot(xb, w1_ref[...], precision=_P,
                              preferred_element_type=jnp.float32) + b1_ref[...]

    wspec = pl.BlockSpec((D, D), lambda i: (0, 0))
    bspec = pl.BlockSpec((1, D), lambda i: (0, 0))
    rspec = pl.BlockSpec((R, D), lambda i: (i, 0))
    return pl.pallas_call(
        body,
        grid=(G,),
        in_specs=[rspec, wspec, bspec, wspec, bspec],
        out_specs=[rspec, rspec],
        out_shape=[jax.ShapeDtypeStruct((N, D), jnp.float32)] * 2,
    )(x, w0, b0, w1, b1)


def _combine(x0, p):
    def body(x0_ref, p_ref, y_ref, s_ref):
        i = pl.program_id(0)
        y = x0_ref[...] + p_ref[0] + p_ref[1]
        y_ref[...] = y
        st = jnp.concatenate(
            [jnp.sum(y, axis=0, keepdims=True),
             jnp.sum(y * y, axis=0, keepdims=True)], axis=0)

        @pl.when(i == 0)
        def _():
            s_ref[...] = st

        @pl.when(i > 0)
        def _():
            s_ref[...] += st

    rspec = pl.BlockSpec((R, D), lambda i: (i, 0))
    return pl.pallas_call(
        body,
        grid=(G,),
        in_specs=[rspec, pl.BlockSpec((2, R, D), lambda i: (0, i, 0))],
        out_specs=[rspec, pl.BlockSpec((2, D), lambda i: (0, 0))],
        out_shape=[jax.ShapeDtypeStruct((N, D), jnp.float32),
                   jax.ShapeDtypeStruct((2, D), jnp.float32)],
    )(x0, p)


def _bn_relu_mm2(y, s, g, be, w0, b0, w1, b1):
    def body(y_ref, s_ref, g_ref, be_ref, w0_ref, b0_ref, w1_ref, b1_ref,
             o0_ref, o1_ref):
        mu = s_ref[0:1, :] * (1.0 / N)
        ex2 = s_ref[1:2, :] * (1.0 / N)
        inv = lax.rsqrt(ex2 - mu * mu + 1e-5)
        xn = jnp.maximum((y_ref[...] - mu) * (inv * g_ref[...]) + be_ref[...], 0.0)
        o0_ref[...] = jnp.dot(xn, w0_ref[...], precision=_P,
                              preferred_element_type=jnp.float32) + b0_ref[...]
        o1_ref[...] = jnp.dot(xn, w1_ref[...], precision=_P,
                              preferred_element_type=jnp.float32) + b1_ref[...]

    wspec = pl.BlockSpec((D, D), lambda i: (0, 0))
    bspec = pl.BlockSpec((1, D), lambda i: (0, 0))
    rspec = pl.BlockSpec((R, D), lambda i: (i, 0))
    return pl.pallas_call(
        body,
        grid=(G,),
        in_specs=[rspec, pl.BlockSpec((2, D), lambda i: (0, 0)),
                  bspec, bspec, wspec, bspec, wspec, bspec],
        out_specs=[rspec, rspec],
        out_shape=[jax.ShapeDtypeStruct((N, D), jnp.float32)] * 2,
    )(y, s, g, be, w0, b0, w1, b1)


def _pool_heads(y, s, g, be, ids3, fw, fb, hw, hb):
    def body(y_ref, s_ref, g_ref, be_ref, ids_ref, fw_ref, fb_ref, hw_ref,
             hb_ref, o_ref, seg_acc, cnt_acc):
        i = pl.program_id(0)

        @pl.when(i == 0)
        def _():
            seg_acc[...] = jnp.zeros_like(seg_acc)
            cnt_acc[...] = jnp.zeros_like(cnt_acc)

        mu = s_ref[0:1, :] * (1.0 / N)
        ex2 = s_ref[1:2, :] * (1.0 / N)
        inv = lax.rsqrt(ex2 - mu * mu + 1e-5)
        xn = jnp.maximum((y_ref[...] - mu) * (inv * g_ref[...]) + be_ref[...], 0.0)
        ids = ids_ref[0]
        onehot = (jnp.broadcast_to(ids, (M, R))
                  == lax.broadcasted_iota(jnp.int32, (M, R), 0)).astype(jnp.float32)
        seg_acc[...] += jnp.dot(onehot, xn, precision=_P,
                                preferred_element_type=jnp.float32)
        cnt_acc[...] += jnp.broadcast_to(
            jnp.sum(onehot, axis=1, keepdims=True), (M, D))

        @pl.when(i == G - 1)
        def _():
            mean = seg_acc[...] / jnp.maximum(cnt_acc[...], 1.0)
            h = jnp.maximum(
                jnp.dot(mean, fw_ref[...], precision=_P,
                        preferred_element_type=jnp.float32) + fb_ref[...], 0.0)
            o_ref[...] = jnp.dot(h, hw_ref[...], precision=_P,
                                 preferred_element_type=jnp.float32) + hb_ref[...]

    rspec = pl.BlockSpec((R, D), lambda i: (i, 0))
    bspec = pl.BlockSpec((1, D), lambda i: (0, 0))
    wspec = pl.BlockSpec((D, D), lambda i: (0, 0))
    return pl.pallas_call(
        body,
        grid=(G,),
        in_specs=[rspec, pl.BlockSpec((2, D), lambda i: (0, 0)), bspec, bspec,
                  pl.BlockSpec((1, 1, R), lambda i: (i, 0, 0)),
                  wspec, bspec, wspec, bspec],
        out_specs=pl.BlockSpec((M, D), lambda i: (0, 0)),
        out_shape=jax.ShapeDtypeStruct((M, D), jnp.float32),
        scratch_shapes=[pltpu.VMEM((M, D), jnp.float32),
                        pltpu.VMEM((M, D), jnp.float32)],
    )(y, s, g, be, ids3, fw, fb, hw, hb)


def kernel(verts, edges, mesh_idx, W0_0, b0_0, W1_0, b1_0, gamma0, beta0,
           W0_1, b0_1, W1_1, b1_1, gamma1, beta1, fc1_w, fc1_b,
           style_w, style_b, sem_w, sem_b, func_w, func_b, aes_w, aes_b):
    e0 = edges[:, 0].astype(jnp.int32)
    e1 = edges[:, 1].astype(jnp.int32)
    npad = EPAD - EPAIR
    ar = jnp.arange(npad, dtype=jnp.int32)
    dst = jnp.concatenate([e0, e1, N + ar % (NP - N)]
                          ).reshape(EPAD // CHUNK, CHUNK)
    src = jnp.concatenate([e1, e0, ar % N]
                          ).reshape(EPAD // CHUNK, CHUNK)

    row = lambda v: v.reshape(1, D)
    ids3 = mesh_idx.astype(jnp.int32).reshape(G, 1, R)
    hw = jnp.pad(jnp.concatenate([style_w, sem_w, func_w, aes_w], axis=1),
                 ((0, 0), (0, D - 14)))
    hb = jnp.pad(jnp.concatenate([style_b, sem_b, func_b, aes_b]).reshape(1, 14),
                 ((0, 0), (0, D - 14)))

    x0, x1 = _mm2(verts, W0_0, row(b0_0), W1_0, row(b1_0))
    p = _sc_edge_scatter(x1, dst, src).reshape(2, NP, D)
    y, s = _combine(x0, p)
    x0, x1 = _bn_relu_mm2(y, s, row(gamma0), row(beta0),
                          W0_1, row(b0_1), W1_1, row(b1_1))
    p = _sc_edge_scatter(x1, dst, src).reshape(2, NP, D)
    y, s = _combine(x0, p)
    out = _pool_heads(y, s, row(gamma1), row(beta1), ids3,
                      fc1_w, row(fc1_b), hw, hb)
    return (out[:, 0:3], out[:, 3:5], out[:, 5:9], out[:, 9:14])


# speedup vs baseline: 11.3342x; 1.0002x over previous
import functools

import jax
import jax.numpy as jnp
from jax import lax
from jax.experimental import pallas as pl
from jax.experimental.pallas import tpu as pltpu
from jax.experimental.pallas import tpu_sc as plsc

N = 10000
D = 128
M = 32
E = 320000
EPAIR = 2 * E
NW = 32
CHUNK = 64
PER_TILE = 20480
EPAD = PER_TILE * NW
NCHUNK = PER_TILE // CHUNK
NP = 10240
ZROWS = NP // 16

R = 1000
G = N // R

_P = None


NBUF = 5
NI = 10
UNROLL = 10


def _sc_edge_scatter(x1, dst2, src2):
    mesh = plsc.VectorSubcoreMesh(core_axis_name="c", subcore_axis_name="s")

    @functools.partial(
        pl.kernel,
        out_type=jax.ShapeDtypeStruct((2 * NP, D), jnp.float32),
        mesh=mesh,
        scratch_types=[
            [pltpu.VMEM((CHUNK,), jnp.int32)] * NI,
            [pltpu.VMEM((CHUNK,), jnp.int32)] * NI,
            [pltpu.VMEM((CHUNK, D), jnp.float32)] * NBUF,
            pltpu.VMEM((8, D), jnp.float32),
            pltpu.VMEM_SHARED((NP, D), jnp.float32),
            [pltpu.SemaphoreType.DMA] * NI,
            [pltpu.SemaphoreType.DMA] * NBUF,
            [pltpu.SemaphoreType.DMA] * NBUF,
        ],
    )
    def body(x1_hbm, dst_hbm, src_hbm, out_hbm, dstv, srcv, rbufs, zbuf, agg,
             isems, rsems, ssems):
        cid = lax.axis_index("c")
        sid = lax.axis_index("s")
        wid = sid * 2 + cid
        base = wid * NCHUNK

        def idx_issue(slot, h):
            pltpu.async_copy(dst_hbm.at[base + h], dstv[slot], isems[slot])
            pltpu.async_copy(src_hbm.at[base + h], srcv[slot], isems[slot])

        def idx_wait(slot, h):
            pltpu.make_async_copy(dst_hbm.at[base + h], dstv[slot],
                                  isems[slot]).wait()
            pltpu.make_async_copy(src_hbm.at[base + h], srcv[slot],
                                  isems[slot]).wait()

        def gather_issue(slot, islot):
            pltpu.async_copy(x1_hbm.at[srcv[islot]], rbufs[slot], rsems[slot])

        def gather_wait(slot, islot):
            pltpu.make_async_copy(x1_hbm.at[srcv[islot]], rbufs[slot],
                                  rsems[slot]).wait()

        for h in range(NI):
            idx_issue(h, h)

        def zrow(r, carry):
            for c8 in range(8):
                zbuf[r, pl.ds(c8 * 16, 16)] = jnp.zeros((16,), jnp.float32)
            return carry

        lax.fori_loop(0, 8, zrow, 0)

        def zcopy(k, carry):
            pltpu.sync_copy(zbuf, agg.at[pl.ds(sid * ZROWS + k * 8, 8)])
            return carry

        lax.fori_loop(0, ZROWS // 8, zcopy, 0)
        plsc.subcore_barrier()

        def scatter_issue(slot, islot):
            pltpu.async_copy(rbufs[slot], agg.at[dstv[islot]], ssems[slot],
                             add=True)

        def scatter_wait(slot, islot):
            pltpu.make_async_copy(rbufs[slot], agg.at[dstv[islot]],
                                  ssems[slot]).wait()

        for h in range(NBUF - 1):
            idx_wait(h, h)
            gather_issue(h, h)

        def step(t, carry):
            for u in range(UNROLL):
                g = t * UNROLL + u
                nx = g + NBUF - 1
                ns = (u + NBUF - 1) % NBUF
                ni = (u + NBUF - 1) % NI

                @pl.when(g >= 1)
                def _():
                    scatter_wait(ns, ni)

                @pl.when(nx < NCHUNK)
                def _():
                    idx_wait(ni, nx)
                    gather_issue(ns, ni)

                gather_wait(u % NBUF, u)
                scatter_issue(u % NBUF, u)

                @pl.when((g >= 1) & (g + NI - 1 < NCHUNK))
                def _():
                    idx_issue((u + NI - 1) % NI, g + NI - 1)
            return carry

        lax.fori_loop(0, NCHUNK // UNROLL, step, 0)
        scatter_wait((NCHUNK - 1) % NBUF, (NCHUNK - 1) % NI)
        plsc.subcore_barrier()
        pltpu.sync_copy(
            agg.at[pl.ds(sid * ZROWS, ZROWS)],
            out_hbm.at[pl.ds(cid * NP + sid * ZROWS, ZROWS)],
        )

    return body(x1, dst2, src2)


def _mm2(x, w0, b0, w1, b1):
    def body(x_ref, w0_ref, b0_ref, w1_ref, b1_ref, o0_ref, o1_ref):
        xb = x_ref[...]
        o0_ref[...] = jnp.dot(xb, w0_ref[...], precision=_P,
                              preferred_element_type=jnp.float32) + b0_ref[...]
        o1_ref[...] = jnp.dot(xb, w1_ref[...], precision=_P,
                              preferred_element_type=jnp.float32) + b1_ref[...]

    wspec = pl.BlockSpec((D, D), lambda i: (0, 0))
    bspec = pl.BlockSpec((1, D), lambda i: (0, 0))
    rspec = pl.BlockSpec((R, D), lambda i: (i, 0))
    return pl.pallas_call(
        body,
        grid=(G,),
        in_specs=[rspec, wspec, bspec, wspec, bspec],
        out_specs=[rspec, rspec],
        out_shape=[jax.ShapeDtypeStruct((N, D), jnp.float32)] * 2,
    )(x, w0, b0, w1, b1)


def _combine(x0, p):
    def body(x0_ref, p_ref, y_ref, s_ref):
        i = pl.program_id(0)
        y = x0_ref[...] + p_ref[0] + p_ref[1]
        y_ref[...] = y
        st = jnp.concatenate(
            [jnp.sum(y, axis=0, keepdims=True),
             jnp.sum(y * y, axis=0, keepdims=True)], axis=0)

        @pl.when(i == 0)
        def _():
            s_ref[...] = st

        @pl.when(i > 0)
        def _():
            s_ref[...] += st

    rspec = pl.BlockSpec((R, D), lambda i: (i, 0))
    return pl.pallas_call(
        body,
        grid=(G,),
        in_specs=[rspec, pl.BlockSpec((2, R, D), lambda i: (0, i, 0))],
        out_specs=[rspec, pl.BlockSpec((2, D), lambda i: (0, 0))],
        out_shape=[jax.ShapeDtypeStruct((N, D), jnp.float32),
                   jax.ShapeDtypeStruct((2, D), jnp.float32)],
    )(x0, p)


def _bn_relu_mm2(y, s, g, be, w0, b0, w1, b1):
    def body(y_ref, s_ref, g_ref, be_ref, w0_ref, b0_ref, w1_ref, b1_ref,
             o0_ref, o1_ref):
        mu = s_ref[0:1, :] * (1.0 / N)
        ex2 = s_ref[1:2, :] * (1.0 / N)
        inv = lax.rsqrt(ex2 - mu * mu + 1e-5)
        xn = jnp.maximum((y_ref[...] - mu) * (inv * g_ref[...]) + be_ref[...], 0.0)
        o0_ref[...] = jnp.dot(xn, w0_ref[...], precision=_P,
                              preferred_element_type=jnp.float32) + b0_ref[...]
        o1_ref[...] = jnp.dot(xn, w1_ref[...], precision=_P,
                              preferred_element_type=jnp.float32) + b1_ref[...]

    wspec = pl.BlockSpec((D, D), lambda i: (0, 0))
    bspec = pl.BlockSpec((1, D), lambda i: (0, 0))
    rspec = pl.BlockSpec((R, D), lambda i: (i, 0))
    return pl.pallas_call(
        body,
        grid=(G,),
        in_specs=[rspec, pl.BlockSpec((2, D), lambda i: (0, 0)),
                  bspec, bspec, wspec, bspec, wspec, bspec],
        out_specs=[rspec, rspec],
        out_shape=[jax.ShapeDtypeStruct((N, D), jnp.float32)] * 2,
    )(y, s, g, be, w0, b0, w1, b1)


def _pool_heads(y, s, g, be, ids3, fw, fb, hw, hb):
    def body(y_ref, s_ref, g_ref, be_ref, ids_ref, fw_ref, fb_ref, hw_ref,
             hb_ref, o_ref, seg_acc, cnt_acc):
        i = pl.program_id(0)

        @pl.when(i == 0)
        def _():
            seg_acc[...] = jnp.zeros_like(seg_acc)
            cnt_acc[...] = jnp.zeros_like(cnt_acc)

        mu = s_ref[0:1, :] * (1.0 / N)
        ex2 = s_ref[1:2, :] * (1.0 / N)
        inv = lax.rsqrt(ex2 - mu * mu + 1e-5)
        xn = jnp.maximum((y_ref[...] - mu) * (inv * g_ref[...]) + be_ref[...], 0.0)
        ids = ids_ref[0]
        onehot = (jnp.broadcast_to(ids, (M, R))
                  == lax.broadcasted_iota(jnp.int32, (M, R), 0)).astype(jnp.float32)
        seg_acc[...] += jnp.dot(onehot, xn, precision=_P,
                                preferred_element_type=jnp.float32)
        cnt_acc[...] += jnp.broadcast_to(
            jnp.sum(onehot, axis=1, keepdims=True), (M, D))

        @pl.when(i == G - 1)
        def _():
            mean = seg_acc[...] / jnp.maximum(cnt_acc[...], 1.0)
            h = jnp.maximum(
                jnp.dot(mean, fw_ref[...], precision=_P,
                        preferred_element_type=jnp.float32) + fb_ref[...], 0.0)
            o_ref[...] = jnp.dot(h, hw_ref[...], precision=_P,
                                 preferred_element_type=jnp.float32) + hb_ref[...]

    rspec = pl.BlockSpec((R, D), lambda i: (i, 0))
    bspec = pl.BlockSpec((1, D), lambda i: (0, 0))
    wspec = pl.BlockSpec((D, D), lambda i: (0, 0))
    return pl.pallas_call(
        body,
        grid=(G,),
        in_specs=[rspec, pl.BlockSpec((2, D), lambda i: (0, 0)), bspec, bspec,
                  pl.BlockSpec((1, 1, R), lambda i: (i, 0, 0)),
                  wspec, bspec, wspec, bspec],
        out_specs=pl.BlockSpec((M, D), lambda i: (0, 0)),
        out_shape=jax.ShapeDtypeStruct((M, D), jnp.float32),
        scratch_shapes=[pltpu.VMEM((M, D), jnp.float32),
                        pltpu.VMEM((M, D), jnp.float32)],
    )(y, s, g, be, ids3, fw, fb, hw, hb)


def kernel(verts, edges, mesh_idx, W0_0, b0_0, W1_0, b1_0, gamma0, beta0,
           W0_1, b0_1, W1_1, b1_1, gamma1, beta1, fc1_w, fc1_b,
           style_w, style_b, sem_w, sem_b, func_w, func_b, aes_w, aes_b):
    e0 = edges[:, 0].astype(jnp.int32)
    e1 = edges[:, 1].astype(jnp.int32)
    npad = EPAD - EPAIR
    ar = jnp.arange(npad, dtype=jnp.int32)
    dst = jnp.concatenate([e0, e1, N + ar % (NP - N)]
                          ).reshape(EPAD // CHUNK, CHUNK)
    src = jnp.concatenate([e1, e0, ar % N]
                          ).reshape(EPAD // CHUNK, CHUNK)

    row = lambda v: v.reshape(1, D)
    ids3 = mesh_idx.astype(jnp.int32).reshape(G, 1, R)
    hw = jnp.pad(jnp.concatenate([style_w, sem_w, func_w, aes_w], axis=1),
                 ((0, 0), (0, D - 14)))
    hb = jnp.pad(jnp.concatenate([style_b, sem_b, func_b, aes_b]).reshape(1, 14),
                 ((0, 0), (0, D - 14)))

    x0, x1 = _mm2(verts, W0_0, row(b0_0), W1_0, row(b1_0))
    p = _sc_edge_scatter(x1, dst, src).reshape(2, NP, D)
    y, s = _combine(x0, p)
    x0, x1 = _bn_relu_mm2(y, s, row(gamma0), row(beta0),
                          W0_1, row(b0_1), W1_1, row(b1_1))
    p = _sc_edge_scatter(x1, dst, src).reshape(2, NP, D)
    y, s = _combine(x0, p)
    out = _pool_heads(y, s, row(gamma1), row(beta1), ids3,
                      fc1_w, row(fc1_b), hw, hb)
    return (out[:, 0:3], out[:, 3:5], out[:, 5:9], out[:, 9:14])


# speedup vs baseline: 11.5583x; 1.0198x over previous
import functools

import jax
import jax.numpy as jnp
from jax import lax
from jax.experimental import pallas as pl
from jax.experimental.pallas import tpu as pltpu
from jax.experimental.pallas import tpu_sc as plsc

N = 10000
D = 128
M = 32
E = 320000
EPAIR = 2 * E
NW = 32
CHUNK = 64
PER_TILE = 20480
EPAD = PER_TILE * NW
NCHUNK = PER_TILE // CHUNK
NP = 10240
ZROWS = NP // 16

R = 1000
G = N // R

_P = None


NBUF = 5
NI = 10
UNROLL = 10


def _sc_edge_scatter(x1, dst2, src2):
    mesh = plsc.VectorSubcoreMesh(core_axis_name="c", subcore_axis_name="s")

    @functools.partial(
        pl.kernel,
        out_type=jax.ShapeDtypeStruct((2 * NP, D), jnp.float32),
        mesh=mesh,
        scratch_types=[
            [pltpu.VMEM((CHUNK,), jnp.int32)] * NI,
            [pltpu.VMEM((CHUNK,), jnp.int32)] * NI,
            [pltpu.VMEM((CHUNK, D), jnp.float32)] * NBUF,
            pltpu.VMEM((8, D), jnp.float32),
            pltpu.VMEM_SHARED((NP, D), jnp.float32),
            [pltpu.SemaphoreType.DMA] * NI,
            [pltpu.SemaphoreType.DMA] * NBUF,
            [pltpu.SemaphoreType.DMA] * NBUF,
        ],
    )
    def body(x1_hbm, dst_hbm, src_hbm, out_hbm, dstv, srcv, rbufs, zbuf, agg,
             isems, rsems, ssems):
        cid = lax.axis_index("c")
        sid = lax.axis_index("s")
        wid = sid * 2 + cid
        base = wid * NCHUNK

        def idx_issue(slot, h):
            pltpu.async_copy(dst_hbm.at[base + h], dstv[slot], isems[slot])
            pltpu.async_copy(src_hbm.at[base + h], srcv[slot], isems[slot])

        def idx_wait(slot, h):
            pltpu.make_async_copy(dst_hbm.at[base + h], dstv[slot],
                                  isems[slot]).wait()
            pltpu.make_async_copy(src_hbm.at[base + h], srcv[slot],
                                  isems[slot]).wait()

        def gather_issue(slot, islot):
            pltpu.async_copy(x1_hbm.at[srcv[islot]], rbufs[slot], rsems[slot])

        def gather_wait(slot, islot):
            pltpu.make_async_copy(x1_hbm.at[srcv[islot]], rbufs[slot],
                                  rsems[slot]).wait()

        for h in range(NI):
            idx_issue(h, h)

        def zrow(r, carry):
            for c8 in range(8):
                zbuf[r, pl.ds(c8 * 16, 16)] = jnp.zeros((16,), jnp.float32)
            return carry

        lax.fori_loop(0, 8, zrow, 0)

        def zcopy(k, carry):
            pltpu.sync_copy(zbuf, agg.at[pl.ds(sid * ZROWS + k * 8, 8)])
            return carry

        lax.fori_loop(0, ZROWS // 8, zcopy, 0)
        plsc.subcore_barrier()

        def scatter_issue(slot, islot):
            pltpu.async_copy(rbufs[slot], agg.at[dstv[islot]], ssems[slot],
                             add=True)

        def scatter_wait(slot, islot):
            pltpu.make_async_copy(rbufs[slot], agg.at[dstv[islot]],
                                  ssems[slot]).wait()

        for h in range(NBUF - 1):
            idx_wait(h, h)
            gather_issue(h, h)

        def step(t, carry):
            for u in range(UNROLL):
                g = t * UNROLL + u
                nx = g + NBUF - 1
                ns = (u + NBUF - 1) % NBUF
                ni = (u + NBUF - 1) % NI

                @pl.when(g >= 1)
                def _():
                    scatter_wait(ns, ni)

                @pl.when(nx < NCHUNK)
                def _():
                    idx_wait(ni, nx)
                    gather_issue(ns, ni)

                gather_wait(u % NBUF, u)
                scatter_issue(u % NBUF, u)

                @pl.when((g >= 1) & (g + NI - 1 < NCHUNK))
                def _():
                    idx_issue((u + NI - 1) % NI, g + NI - 1)
            return carry

        lax.fori_loop(0, NCHUNK // UNROLL, step, 0)
        scatter_wait((NCHUNK - 1) % NBUF, (NCHUNK - 1) % NI)
        plsc.subcore_barrier()
        pltpu.sync_copy(
            agg.at[pl.ds(sid * ZROWS, ZROWS)],
            out_hbm.at[pl.ds(cid * NP + sid * ZROWS, ZROWS)],
        )

    return body(x1, dst2, src2)


def _mm2(x, w0, b0, w1, b1):
    def body(x_ref, w0_ref, b0_ref, w1_ref, b1_ref, o0_ref, o1_ref):
        xb = x_ref[...]
        o0_ref[...] = jnp.dot(xb, w0_ref[...], precision=_P,
                              preferred_element_type=jnp.float32) + b0_ref[...]
        o1_ref[...] = jnp.dot(xb, w1_ref[...], precision=_P,
                              preferred_element_type=jnp.float32) + b1_ref[...]

    wspec = pl.BlockSpec((D, D), lambda i: (0, 0))
    bspec = pl.BlockSpec((1, D), lambda i: (0, 0))
    rspec = pl.BlockSpec((R, D), lambda i: (i, 0))
    return pl.pallas_call(
        body,
        grid=(G,),
        in_specs=[rspec, wspec, bspec, wspec, bspec],
        out_specs=[rspec, rspec],
        out_shape=[jax.ShapeDtypeStruct((N, D), jnp.float32)] * 2,
    )(x, w0, b0, w1, b1)


def _bn_stats(y):
    return jnp.concatenate(
        [jnp.sum(y, axis=0, keepdims=True),
         jnp.sum(y * y, axis=0, keepdims=True)], axis=0)


def _bn_relu(y, s_ref, g_ref, be_ref):
    mu = s_ref[0:1, :] * (1.0 / N)
    ex2 = s_ref[1:2, :] * (1.0 / N)
    inv = lax.rsqrt(ex2 - mu * mu + 1e-5)
    return jnp.maximum((y - mu) * (inv * g_ref[...]) + be_ref[...], 0.0)


def _combine_bn_mm2(x0, p, g, be, w0, b0, w1, b1):
    def body(x0_ref, p_ref, g_ref, be_ref, w0_ref, b0_ref, w1_ref, b1_ref,
             o0_ref, o1_ref, y_s, st_s):
        ph = pl.program_id(0)
        i = pl.program_id(1)

        @pl.when(ph == 0)
        def _():
            y = x0_ref[...] + p_ref[0] + p_ref[1]
            y_s[pl.ds(i * R, R), :] = y

            @pl.when(i == 0)
            def _():
                st_s[...] = jnp.zeros_like(st_s)

            st_s[...] += _bn_stats(y)

        @pl.when(ph == 1)
        def _():
            xn = _bn_relu(y_s[pl.ds(i * R, R), :], st_s, g_ref, be_ref)
            o0_ref[...] = jnp.dot(xn, w0_ref[...], precision=_P,
                                  preferred_element_type=jnp.float32) + b0_ref[...]
            o1_ref[...] = jnp.dot(xn, w1_ref[...], precision=_P,
                                  preferred_element_type=jnp.float32) + b1_ref[...]

    wspec = pl.BlockSpec((D, D), lambda ph, i: (0, 0))
    bspec = pl.BlockSpec((1, D), lambda ph, i: (0, 0))
    rspec0 = pl.BlockSpec((R, D), lambda ph, i: (i * (1 - ph), 0))
    rspec1 = pl.BlockSpec((R, D), lambda ph, i: (i * ph, 0))
    return pl.pallas_call(
        body,
        grid=(2, G),
        in_specs=[rspec0, pl.BlockSpec((2, R, D), lambda ph, i: (0, i * (1 - ph), 0)),
                  bspec, bspec, wspec, bspec, wspec, bspec],
        out_specs=[rspec1, rspec1],
        out_shape=[jax.ShapeDtypeStruct((N, D), jnp.float32)] * 2,
        scratch_shapes=[pltpu.VMEM((N, D), jnp.float32),
                        pltpu.VMEM((2, D), jnp.float32)],
    )(x0, p, g, be, w0, b0, w1, b1)


def _combine_bn_pool_heads(x0, p, g, be, ids3, fw, fb, hw, hb):
    def body(x0_ref, p_ref, g_ref, be_ref, ids_ref, fw_ref, fb_ref, hw_ref,
             hb_ref, o_ref, y_s, st_s, seg_s, cnt_s):
        ph = pl.program_id(0)
        i = pl.program_id(1)

        @pl.when(ph == 0)
        def _():
            y = x0_ref[...] + p_ref[0] + p_ref[1]
            y_s[pl.ds(i * R, R), :] = y

            @pl.when(i == 0)
            def _():
                st_s[...] = jnp.zeros_like(st_s)
                seg_s[...] = jnp.zeros_like(seg_s)
                cnt_s[...] = jnp.zeros_like(cnt_s)

            st_s[...] += _bn_stats(y)

        @pl.when(ph == 1)
        def _():
            xn = _bn_relu(y_s[pl.ds(i * R, R), :], st_s, g_ref, be_ref)
            ids = ids_ref[0]
            onehot = (jnp.broadcast_to(ids, (M, R))
                      == lax.broadcasted_iota(jnp.int32, (M, R), 0)
                      ).astype(jnp.float32)
            seg_s[...] += jnp.dot(onehot, xn, precision=_P,
                                  preferred_element_type=jnp.float32)
            cnt_s[...] += jnp.broadcast_to(
                jnp.sum(onehot, axis=1, keepdims=True), (M, D))

            @pl.when(i == G - 1)
            def _():
                mean = seg_s[...] / jnp.maximum(cnt_s[...], 1.0)
                h = jnp.maximum(
                    jnp.dot(mean, fw_ref[...], precision=_P,
                            preferred_element_type=jnp.float32) + fb_ref[...],
                    0.0)
                o_ref[...] = jnp.dot(h, hw_ref[...], precision=_P,
                                     preferred_element_type=jnp.float32
                                     ) + hb_ref[...]

    bspec = pl.BlockSpec((1, D), lambda ph, i: (0, 0))
    wspec = pl.BlockSpec((D, D), lambda ph, i: (0, 0))
    rspec0 = pl.BlockSpec((R, D), lambda ph, i: (i * (1 - ph), 0))
    return pl.pallas_call(
        body,
        grid=(2, G),
        in_specs=[rspec0, pl.BlockSpec((2, R, D), lambda ph, i: (0, i * (1 - ph), 0)),
                  bspec, bspec,
                  pl.BlockSpec((1, 1, R), lambda ph, i: (i * ph, 0, 0)),
                  wspec, bspec, wspec, bspec],
        out_specs=pl.BlockSpec((M, D), lambda ph, i: (0, 0)),
        out_shape=jax.ShapeDtypeStruct((M, D), jnp.float32),
        scratch_shapes=[pltpu.VMEM((N, D), jnp.float32),
                        pltpu.VMEM((2, D), jnp.float32),
                        pltpu.VMEM((M, D), jnp.float32),
                        pltpu.VMEM((M, D), jnp.float32)],
    )(x0, p, g, be, ids3, fw, fb, hw, hb)


def kernel(verts, edges, mesh_idx, W0_0, b0_0, W1_0, b1_0, gamma0, beta0,
           W0_1, b0_1, W1_1, b1_1, gamma1, beta1, fc1_w, fc1_b,
           style_w, style_b, sem_w, sem_b, func_w, func_b, aes_w, aes_b):
    e0 = edges[:, 0].astype(jnp.int32)
    e1 = edges[:, 1].astype(jnp.int32)
    npad = EPAD - EPAIR
    ar = jnp.arange(npad, dtype=jnp.int32)
    dst = jnp.concatenate([e0, e1, N + ar % (NP - N)]
                          ).reshape(EPAD // CHUNK, CHUNK)
    src = jnp.concatenate([e1, e0, ar % N]
                          ).reshape(EPAD // CHUNK, CHUNK)

    row = lambda v: v.reshape(1, D)
    ids3 = mesh_idx.astype(jnp.int32).reshape(G, 1, R)
    hw = jnp.pad(jnp.concatenate([style_w, sem_w, func_w, aes_w], axis=1),
                 ((0, 0), (0, D - 14)))
    hb = jnp.pad(jnp.concatenate([style_b, sem_b, func_b, aes_b]).reshape(1, 14),
                 ((0, 0), (0, D - 14)))

    x0, x1 = _mm2(verts, W0_0, row(b0_0), W1_0, row(b1_0))
    p = _sc_edge_scatter(x1, dst, src).reshape(2, NP, D)
    x0, x1 = _combine_bn_mm2(x0, p, row(gamma0), row(beta0),
                             W0_1, row(b0_1), W1_1, row(b1_1))
    p = _sc_edge_scatter(x1, dst, src).reshape(2, NP, D)
    out = _combine_bn_pool_heads(x0, p, row(gamma1), row(beta1), ids3,
                                 fc1_w, row(fc1_b), hw, hb)
    return (out[:, 0:3], out[:, 3:5], out[:, 5:9], out[:, 9:14])


# speedup vs baseline: 11.7881x; 1.0199x over previous
import functools

import jax
import jax.numpy as jnp
from jax import lax
from jax.experimental import pallas as pl
from jax.experimental.pallas import tpu as pltpu
from jax.experimental.pallas import tpu_sc as plsc

N = 10000
D = 128
M = 32
E = 320000
EPAIR = 2 * E
NW = 32
CHUNK = 64
PER_TILE = 20480
EPAD = PER_TILE * NW
NCHUNK = PER_TILE // CHUNK
NP = 10240
ZROWS = NP // 16

R = 1000
G = N // R

_P = None


NBUF = 5
NI = 10
UNROLL = 10


def _sc_edge_scatter(x1, dst2, src2):
    mesh = plsc.VectorSubcoreMesh(core_axis_name="c", subcore_axis_name="s")

    @functools.partial(
        pl.kernel,
        out_type=jax.ShapeDtypeStruct((2 * NP, D), jnp.float32),
        mesh=mesh,
        scratch_types=[
            [pltpu.VMEM((CHUNK,), jnp.int32)] * NI,
            [pltpu.VMEM((CHUNK,), jnp.int32)] * NI,
            [pltpu.VMEM((CHUNK, D), jnp.float32)] * NBUF,
            pltpu.VMEM_SHARED((NP, D), jnp.float32),
            [pltpu.SemaphoreType.DMA] * NI,
            [pltpu.SemaphoreType.DMA] * NBUF,
            [pltpu.SemaphoreType.DMA] * NBUF,
            pltpu.SemaphoreType.DMA,
        ],
    )
    def body(x1_hbm, dst_hbm, src_hbm, out_hbm, dstv, srcv, rbufs, agg,
             isems, rsems, ssems, zsem):
        cid = lax.axis_index("c")
        sid = lax.axis_index("s")
        wid = sid * 2 + cid
        base = wid * NCHUNK

        def idx_issue(slot, h):
            pltpu.async_copy(dst_hbm.at[base + h], dstv[slot], isems[slot])
            pltpu.async_copy(src_hbm.at[base + h], srcv[slot], isems[slot])

        def idx_wait(slot, h):
            pltpu.make_async_copy(dst_hbm.at[base + h], dstv[slot],
                                  isems[slot]).wait()
            pltpu.make_async_copy(src_hbm.at[base + h], srcv[slot],
                                  isems[slot]).wait()

        def gather_issue(slot, islot):
            pltpu.async_copy(x1_hbm.at[srcv[islot]], rbufs[slot], rsems[slot])

        def gather_wait(slot, islot):
            pltpu.make_async_copy(x1_hbm.at[srcv[islot]], rbufs[slot],
                                  rsems[slot]).wait()

        for h in range(NI):
            idx_issue(h, h)

        def zrow(r, carry):
            for c8 in range(8):
                rbufs[0][r, pl.ds(c8 * 16, 16)] = jnp.zeros((16,), jnp.float32)
            return carry

        lax.fori_loop(0, CHUNK, zrow, 0)
        for k in range(ZROWS // CHUNK):
            pltpu.async_copy(rbufs[0],
                             agg.at[pl.ds(sid * ZROWS + k * CHUNK, CHUNK)],
                             zsem)
        for k in range(ZROWS // CHUNK):
            pltpu.make_async_copy(
                rbufs[0], agg.at[pl.ds(sid * ZROWS + k * CHUNK, CHUNK)],
                zsem).wait()

        def scatter_issue(slot, islot):
            pltpu.async_copy(rbufs[slot], agg.at[dstv[islot]], ssems[slot],
                             add=True)

        def scatter_wait(slot, islot):
            pltpu.make_async_copy(rbufs[slot], agg.at[dstv[islot]],
                                  ssems[slot]).wait()

        for h in range(NBUF - 1):
            idx_wait(h, h)
            gather_issue(h, h)
        plsc.subcore_barrier()

        def step(t, carry):
            for u in range(UNROLL):
                g = t * UNROLL + u
                nx = g + NBUF - 1
                ns = (u + NBUF - 1) % NBUF
                ni = (u + NBUF - 1) % NI

                @pl.when(g >= 1)
                def _():
                    scatter_wait(ns, ni)

                @pl.when(nx < NCHUNK)
                def _():
                    idx_wait(ni, nx)
                    gather_issue(ns, ni)

                gather_wait(u % NBUF, u)
                scatter_issue(u % NBUF, u)

                @pl.when((g >= 1) & (g + NI - 1 < NCHUNK))
                def _():
                    idx_issue((u + NI - 1) % NI, g + NI - 1)
            return carry

        lax.fori_loop(0, NCHUNK // UNROLL, step, 0)
        scatter_wait((NCHUNK - 1) % NBUF, (NCHUNK - 1) % NI)
        plsc.subcore_barrier()
        pltpu.sync_copy(
            agg.at[pl.ds(sid * ZROWS, ZROWS)],
            out_hbm.at[pl.ds(cid * NP + sid * ZROWS, ZROWS)],
        )

    return body(x1, dst2, src2)


def _mm2(x, w0, b0, w1, b1):
    def body(x_ref, w0_ref, b0_ref, w1_ref, b1_ref, o0_ref, o1_ref):
        xb = x_ref[...]
        o0_ref[...] = jnp.dot(xb, w0_ref[...], precision=_P,
                              preferred_element_type=jnp.float32) + b0_ref[...]
        o1_ref[...] = jnp.dot(xb, w1_ref[...], precision=_P,
                              preferred_element_type=jnp.float32) + b1_ref[...]

    wspec = pl.BlockSpec((D, D), lambda i: (0, 0))
    bspec = pl.BlockSpec((1, D), lambda i: (0, 0))
    rspec = pl.BlockSpec((R, D), lambda i: (i, 0))
    return pl.pallas_call(
        body,
        grid=(G,),
        in_specs=[rspec, wspec, bspec, wspec, bspec],
        out_specs=[rspec, rspec],
        out_shape=[jax.ShapeDtypeStruct((N, D), jnp.float32)] * 2,
    )(x, w0, b0, w1, b1)


def _bn_stats(y):
    return jnp.concatenate(
        [jnp.sum(y, axis=0, keepdims=True),
         jnp.sum(y * y, axis=0, keepdims=True)], axis=0)


def _bn_relu(y, s_ref, g_ref, be_ref):
    mu = s_ref[0:1, :] * (1.0 / N)
    ex2 = s_ref[1:2, :] * (1.0 / N)
    inv = lax.rsqrt(ex2 - mu * mu + 1e-5)
    return jnp.maximum((y - mu) * (inv * g_ref[...]) + be_ref[...], 0.0)


def _combine_bn_mm2(x0, p, g, be, w0, b0, w1, b1):
    def body(x0_ref, p_ref, g_ref, be_ref, w0_ref, b0_ref, w1_ref, b1_ref,
             o0_ref, o1_ref, y_s, st_s):
        ph = pl.program_id(0)
        i = pl.program_id(1)

        @pl.when(ph == 0)
        def _():
            y = x0_ref[...] + p_ref[0] + p_ref[1]
            y_s[pl.ds(i * R, R), :] = y

            @pl.when(i == 0)
            def _():
                st_s[...] = jnp.zeros_like(st_s)

            st_s[...] += _bn_stats(y)

        @pl.when(ph == 1)
        def _():
            xn = _bn_relu(y_s[pl.ds(i * R, R), :], st_s, g_ref, be_ref)
            o0_ref[...] = jnp.dot(xn, w0_ref[...], precision=_P,
                                  preferred_element_type=jnp.float32) + b0_ref[...]
            o1_ref[...] = jnp.dot(xn, w1_ref[...], precision=_P,
                                  preferred_element_type=jnp.float32) + b1_ref[...]

    wspec = pl.BlockSpec((D, D), lambda ph, i: (0, 0))
    bspec = pl.BlockSpec((1, D), lambda ph, i: (0, 0))
    rspec0 = pl.BlockSpec((R, D), lambda ph, i: (i * (1 - ph), 0))
    rspec1 = pl.BlockSpec((R, D), lambda ph, i: (i * ph, 0))
    return pl.pallas_call(
        body,
        grid=(2, G),
        in_specs=[rspec0, pl.BlockSpec((2, R, D), lambda ph, i: (0, i * (1 - ph), 0)),
                  bspec, bspec, wspec, bspec, wspec, bspec],
        out_specs=[rspec1, rspec1],
        out_shape=[jax.ShapeDtypeStruct((N, D), jnp.float32)] * 2,
        scratch_shapes=[pltpu.VMEM((N, D), jnp.float32),
                        pltpu.VMEM((2, D), jnp.float32)],
    )(x0, p, g, be, w0, b0, w1, b1)


def _combine_bn_pool_heads(x0, p, g, be, ids3, fw, fb, hw, hb):
    def body(x0_ref, p_ref, g_ref, be_ref, ids_ref, fw_ref, fb_ref, hw_ref,
             hb_ref, o_ref, y_s, st_s, seg_s, cnt_s):
        ph = pl.program_id(0)
        i = pl.program_id(1)

        @pl.when(ph == 0)
        def _():
            y = x0_ref[...] + p_ref[0] + p_ref[1]
            y_s[pl.ds(i * R, R), :] = y

            @pl.when(i == 0)
            def _():
                st_s[...] = jnp.zeros_like(st_s)
                seg_s[...] = jnp.zeros_like(seg_s)
                cnt_s[...] = jnp.zeros_like(cnt_s)

            st_s[...] += _bn_stats(y)

        @pl.when(ph == 1)
        def _():
            xn = _bn_relu(y_s[pl.ds(i * R, R), :], st_s, g_ref, be_ref)
            ids = ids_ref[0]
            onehot = (jnp.broadcast_to(ids, (M, R))
                      == lax.broadcasted_iota(jnp.int32, (M, R), 0)
                      ).astype(jnp.float32)
            seg_s[...] += jnp.dot(onehot, xn, precision=_P,
                                  preferred_element_type=jnp.float32)
            cnt_s[...] += jnp.broadcast_to(
                jnp.sum(onehot, axis=1, keepdims=True), (M, D))

            @pl.when(i == G - 1)
            def _():
                mean = seg_s[...] / jnp.maximum(cnt_s[...], 1.0)
                h = jnp.maximum(
                    jnp.dot(mean, fw_ref[...], precision=_P,
                            preferred_element_type=jnp.float32) + fb_ref[...],
                    0.0)
                o_ref[...] = jnp.dot(h, hw_ref[...], precision=_P,
                                     preferred_element_type=jnp.float32
                                     ) + hb_ref[...]

    bspec = pl.BlockSpec((1, D), lambda ph, i: (0, 0))
    wspec = pl.BlockSpec((D, D), lambda ph, i: (0, 0))
    rspec0 = pl.BlockSpec((R, D), lambda ph, i: (i * (1 - ph), 0))
    return pl.pallas_call(
        body,
        grid=(2, G),
        in_specs=[rspec0, pl.BlockSpec((2, R, D), lambda ph, i: (0, i * (1 - ph), 0)),
                  bspec, bspec,
                  pl.BlockSpec((1, 1, R), lambda ph, i: (i * ph, 0, 0)),
                  wspec, bspec, wspec, bspec],
        out_specs=pl.BlockSpec((M, D), lambda ph, i: (0, 0)),
        out_shape=jax.ShapeDtypeStruct((M, D), jnp.float32),
        scratch_shapes=[pltpu.VMEM((N, D), jnp.float32),
                        pltpu.VMEM((2, D), jnp.float32),
                        pltpu.VMEM((M, D), jnp.float32),
                        pltpu.VMEM((M, D), jnp.float32)],
    )(x0, p, g, be, ids3, fw, fb, hw, hb)


def kernel(verts, edges, mesh_idx, W0_0, b0_0, W1_0, b1_0, gamma0, beta0,
           W0_1, b0_1, W1_1, b1_1, gamma1, beta1, fc1_w, fc1_b,
           style_w, style_b, sem_w, sem_b, func_w, func_b, aes_w, aes_b):
    e0 = edges[:, 0].astype(jnp.int32)
    e1 = edges[:, 1].astype(jnp.int32)
    npad = EPAD - EPAIR
    ar = jnp.arange(npad, dtype=jnp.int32)
    dst = jnp.concatenate([e0, e1, N + ar % (NP - N)]
                          ).reshape(EPAD // CHUNK, CHUNK)
    src = jnp.concatenate([e1, e0, ar % N]
                          ).reshape(EPAD // CHUNK, CHUNK)

    row = lambda v: v.reshape(1, D)
    ids3 = mesh_idx.astype(jnp.int32).reshape(G, 1, R)
    hw = jnp.pad(jnp.concatenate([style_w, sem_w, func_w, aes_w], axis=1),
                 ((0, 0), (0, D - 14)))
    hb = jnp.pad(jnp.concatenate([style_b, sem_b, func_b, aes_b]).reshape(1, 14),
                 ((0, 0), (0, D - 14)))

    x0, x1 = _mm2(verts, W0_0, row(b0_0), W1_0, row(b1_0))
    p = _sc_edge_scatter(x1, dst, src).reshape(2, NP, D)
    x0, x1 = _combine_bn_mm2(x0, p, row(gamma0), row(beta0),
                             W0_1, row(b0_1), W1_1, row(b1_1))
    p = _sc_edge_scatter(x1, dst, src).reshape(2, NP, D)
    out = _combine_bn_pool_heads(x0, p, row(gamma1), row(beta1), ids3,
                                 fc1_w, row(fc1_b), hw, hb)
    return (out[:, 0:3], out[:, 3:5], out[:, 5:9], out[:, 9:14])


# speedup vs baseline: 12.1330x; 1.0293x over previous
import functools

import jax
import jax.numpy as jnp
from jax import lax
from jax.experimental import pallas as pl
from jax.experimental.pallas import tpu as pltpu
from jax.experimental.pallas import tpu_sc as plsc

N = 10000
D = 128
M = 32
E = 320000
EPAIR = 2 * E
NW = 32
CHUNK = 64
PER_TILE = 20480
EPAD = PER_TILE * NW
NCHUNK = PER_TILE // CHUNK
NP = 10240
ZROWS = NP // 16

R = 2000
G = N // R

_P = None


NBUF = 5
NI = 10
UNROLL = 10


def _sc_edge_scatter(x1, dst2, src2):
    mesh = plsc.VectorSubcoreMesh(core_axis_name="c", subcore_axis_name="s")

    @functools.partial(
        pl.kernel,
        out_type=jax.ShapeDtypeStruct((2 * NP, D), jnp.float32),
        mesh=mesh,
        scratch_types=[
            [pltpu.VMEM((CHUNK,), jnp.int32)] * NI,
            [pltpu.VMEM((CHUNK,), jnp.int32)] * NI,
            [pltpu.VMEM((CHUNK, D), jnp.float32)] * NBUF,
            pltpu.VMEM_SHARED((NP, D), jnp.float32),
            [pltpu.SemaphoreType.DMA] * NI,
            [pltpu.SemaphoreType.DMA] * NBUF,
            [pltpu.SemaphoreType.DMA] * NBUF,
            pltpu.SemaphoreType.DMA,
        ],
    )
    def body(x1_hbm, dst_hbm, src_hbm, out_hbm, dstv, srcv, rbufs, agg,
             isems, rsems, ssems, zsem):
        cid = lax.axis_index("c")
        sid = lax.axis_index("s")
        wid = sid * 2 + cid
        base = wid * NCHUNK

        def idx_issue(slot, h):
            pltpu.async_copy(dst_hbm.at[base + h], dstv[slot], isems[slot])
            pltpu.async_copy(src_hbm.at[base + h], srcv[slot], isems[slot])

        def idx_wait(slot, h):
            pltpu.make_async_copy(dst_hbm.at[base + h], dstv[slot],
                                  isems[slot]).wait()
            pltpu.make_async_copy(src_hbm.at[base + h], srcv[slot],
                                  isems[slot]).wait()

        def gather_issue(slot, islot):
            pltpu.async_copy(x1_hbm.at[srcv[islot]], rbufs[slot], rsems[slot])

        def gather_wait(slot, islot):
            pltpu.make_async_copy(x1_hbm.at[srcv[islot]], rbufs[slot],
                                  rsems[slot]).wait()

        for h in range(NI):
            idx_issue(h, h)

        def zrow(r, carry):
            for c8 in range(8):
                rbufs[0][r, pl.ds(c8 * 16, 16)] = jnp.zeros((16,), jnp.float32)
            return carry

        lax.fori_loop(0, CHUNK, zrow, 0)
        for k in range(ZROWS // CHUNK):
            pltpu.async_copy(rbufs[0],
                             agg.at[pl.ds(sid * ZROWS + k * CHUNK, CHUNK)],
                             zsem)
        for k in range(ZROWS // CHUNK):
            pltpu.make_async_copy(
                rbufs[0], agg.at[pl.ds(sid * ZROWS + k * CHUNK, CHUNK)],
                zsem).wait()

        def scatter_issue(slot, islot):
            pltpu.async_copy(rbufs[slot], agg.at[dstv[islot]], ssems[slot],
                             add=True)

        def scatter_wait(slot, islot):
            pltpu.make_async_copy(rbufs[slot], agg.at[dstv[islot]],
                                  ssems[slot]).wait()

        for h in range(NBUF - 1):
            idx_wait(h, h)
            gather_issue(h, h)
        plsc.subcore_barrier()

        def step(t, carry):
            for u in range(UNROLL):
                g = t * UNROLL + u
                nx = g + NBUF - 1
                ns = (u + NBUF - 1) % NBUF
                ni = (u + NBUF - 1) % NI

                @pl.when(g >= 1)
                def _():
                    scatter_wait(ns, ni)

                @pl.when(nx < NCHUNK)
                def _():
                    idx_wait(ni, nx)
                    gather_issue(ns, ni)

                gather_wait(u % NBUF, u)
                scatter_issue(u % NBUF, u)

                @pl.when((g >= 1) & (g + NI - 1 < NCHUNK))
                def _():
                    idx_issue((u + NI - 1) % NI, g + NI - 1)
            return carry

        lax.fori_loop(0, NCHUNK // UNROLL, step, 0)
        scatter_wait((NCHUNK - 1) % NBUF, (NCHUNK - 1) % NI)
        plsc.subcore_barrier()
        pltpu.sync_copy(
            agg.at[pl.ds(sid * ZROWS, ZROWS)],
            out_hbm.at[pl.ds(cid * NP + sid * ZROWS, ZROWS)],
        )

    return body(x1, dst2, src2)


def _mm2(x, w0, b0, w1, b1):
    def body(x_ref, w0_ref, b0_ref, w1_ref, b1_ref, o0_ref, o1_ref):
        xb = x_ref[...]
        o0_ref[...] = jnp.dot(xb, w0_ref[...], precision=_P,
                              preferred_element_type=jnp.float32) + b0_ref[...]
        o1_ref[...] = jnp.dot(xb, w1_ref[...], precision=_P,
                              preferred_element_type=jnp.float32) + b1_ref[...]

    wspec = pl.BlockSpec((D, D), lambda i: (0, 0))
    bspec = pl.BlockSpec((1, D), lambda i: (0, 0))
    rspec = pl.BlockSpec((R, D), lambda i: (i, 0))
    return pl.pallas_call(
        body,
        grid=(G,),
        in_specs=[rspec, wspec, bspec, wspec, bspec],
        out_specs=[rspec, rspec],
        out_shape=[jax.ShapeDtypeStruct((N, D), jnp.float32)] * 2,
    )(x, w0, b0, w1, b1)


def _bn_stats(y):
    return jnp.concatenate(
        [jnp.sum(y, axis=0, keepdims=True),
         jnp.sum(y * y, axis=0, keepdims=True)], axis=0)


def _bn_relu(y, s_ref, g_ref, be_ref):
    mu = s_ref[0:1, :] * (1.0 / N)
    ex2 = s_ref[1:2, :] * (1.0 / N)
    inv = lax.rsqrt(ex2 - mu * mu + 1e-5)
    return jnp.maximum((y - mu) * (inv * g_ref[...]) + be_ref[...], 0.0)


def _combine_bn_mm2(x0, p, g, be, w0, b0, w1, b1):
    def body(x0_ref, p_ref, g_ref, be_ref, w0_ref, b0_ref, w1_ref, b1_ref,
             o0_ref, o1_ref, y_s, st_s):
        ph = pl.program_id(0)
        i = pl.program_id(1)

        @pl.when(ph == 0)
        def _():
            y = x0_ref[...] + p_ref[0] + p_ref[1]
            y_s[pl.ds(i * R, R), :] = y

            @pl.when(i == 0)
            def _():
                st_s[...] = jnp.zeros_like(st_s)

            st_s[...] += _bn_stats(y)

        @pl.when(ph == 1)
        def _():
            xn = _bn_relu(y_s[pl.ds(i * R, R), :], st_s, g_ref, be_ref)
            o0_ref[...] = jnp.dot(xn, w0_ref[...], precision=_P,
                                  preferred_element_type=jnp.float32) + b0_ref[...]
            o1_ref[...] = jnp.dot(xn, w1_ref[...], precision=_P,
                                  preferred_element_type=jnp.float32) + b1_ref[...]

    wspec = pl.BlockSpec((D, D), lambda ph, i: (0, 0))
    bspec = pl.BlockSpec((1, D), lambda ph, i: (0, 0))
    rspec0 = pl.BlockSpec((R, D), lambda ph, i: (i * (1 - ph), 0))
    rspec1 = pl.BlockSpec((R, D), lambda ph, i: (i * ph, 0))
    return pl.pallas_call(
        body,
        grid=(2, G),
        in_specs=[rspec0, pl.BlockSpec((2, R, D), lambda ph, i: (0, i * (1 - ph), 0)),
                  bspec, bspec, wspec, bspec, wspec, bspec],
        out_specs=[rspec1, rspec1],
        out_shape=[jax.ShapeDtypeStruct((N, D), jnp.float32)] * 2,
        scratch_shapes=[pltpu.VMEM((N, D), jnp.float32),
                        pltpu.VMEM((2, D), jnp.float32)],
    )(x0, p, g, be, w0, b0, w1, b1)


def _combine_bn_pool_heads(x0, p, g, be, ids3, fw, fb, hw, hb):
    def body(x0_ref, p_ref, g_ref, be_ref, ids_ref, fw_ref, fb_ref, hw_ref,
             hb_ref, o_ref, y_s, st_s, seg_s, cnt_s):
        ph = pl.program_id(0)
        i = pl.program_id(1)

        @pl.when(ph == 0)
        def _():
            y = x0_ref[...] + p_ref[0] + p_ref[1]
            y_s[pl.ds(i * R, R), :] = y

            @pl.when(i == 0)
            def _():
                st_s[...] = jnp.zeros_like(st_s)
                seg_s[...] = jnp.zeros_like(seg_s)
                cnt_s[...] = jnp.zeros_like(cnt_s)

            st_s[...] += _bn_stats(y)

        @pl.when(ph == 1)
        def _():
            xn = _bn_relu(y_s[pl.ds(i * R, R), :], st_s, g_ref, be_ref)
            ids = ids_ref[0]
            onehot = (jnp.broadcast_to(ids, (M, R))
                      == lax.broadcasted_iota(jnp.int32, (M, R), 0)
                      ).astype(jnp.float32)
            seg_s[...] += jnp.dot(onehot, xn, precision=_P,
                                  preferred_element_type=jnp.float32)
            cnt_s[...] += jnp.broadcast_to(
                jnp.sum(onehot, axis=1, keepdims=True), (M, D))

            @pl.when(i == G - 1)
            def _():
                mean = seg_s[...] / jnp.maximum(cnt_s[...], 1.0)
                h = jnp.maximum(
                    jnp.dot(mean, fw_ref[...], precision=_P,
                            preferred_element_type=jnp.float32) + fb_ref[...],
                    0.0)
                o_ref[...] = jnp.dot(h, hw_ref[...], precision=_P,
                                     preferred_element_type=jnp.float32
                                     ) + hb_ref[...]

    bspec = pl.BlockSpec((1, D), lambda ph, i: (0, 0))
    wspec = pl.BlockSpec((D, D), lambda ph, i: (0, 0))
    rspec0 = pl.BlockSpec((R, D), lambda ph, i: (i * (1 - ph), 0))
    return pl.pallas_call(
        body,
        grid=(2, G),
        in_specs=[rspec0, pl.BlockSpec((2, R, D), lambda ph, i: (0, i * (1 - ph), 0)),
                  bspec, bspec,
                  pl.BlockSpec((1, 1, R), lambda ph, i: (i * ph, 0, 0)),
                  wspec, bspec, wspec, bspec],
        out_specs=pl.BlockSpec((M, D), lambda ph, i: (0, 0)),
        out_shape=jax.ShapeDtypeStruct((M, D), jnp.float32),
        scratch_shapes=[pltpu.VMEM((N, D), jnp.float32),
                        pltpu.VMEM((2, D), jnp.float32),
                        pltpu.VMEM((M, D), jnp.float32),
                        pltpu.VMEM((M, D), jnp.float32)],
    )(x0, p, g, be, ids3, fw, fb, hw, hb)


def kernel(verts, edges, mesh_idx, W0_0, b0_0, W1_0, b1_0, gamma0, beta0,
           W0_1, b0_1, W1_1, b1_1, gamma1, beta1, fc1_w, fc1_b,
           style_w, style_b, sem_w, sem_b, func_w, func_b, aes_w, aes_b):
    e0 = edges[:, 0].astype(jnp.int32)
    e1 = edges[:, 1].astype(jnp.int32)
    npad = EPAD - EPAIR
    ar = jnp.arange(npad, dtype=jnp.int32)
    dst = jnp.concatenate([e0, e1, N + ar % (NP - N)]
                          ).reshape(EPAD // CHUNK, CHUNK)
    src = jnp.concatenate([e1, e0, ar % N]
                          ).reshape(EPAD // CHUNK, CHUNK)

    row = lambda v: v.reshape(1, D)
    ids3 = mesh_idx.astype(jnp.int32).reshape(G, 1, R)
    hw = jnp.pad(jnp.concatenate([style_w, sem_w, func_w, aes_w], axis=1),
                 ((0, 0), (0, D - 14)))
    hb = jnp.pad(jnp.concatenate([style_b, sem_b, func_b, aes_b]).reshape(1, 14),
                 ((0, 0), (0, D - 14)))

    x0, x1 = _mm2(verts, W0_0, row(b0_0), W1_0, row(b1_0))
    p = _sc_edge_scatter(x1, dst, src).reshape(2, NP, D)
    x0, x1 = _combine_bn_mm2(x0, p, row(gamma0), row(beta0),
                             W0_1, row(b0_1), W1_1, row(b1_1))
    p = _sc_edge_scatter(x1, dst, src).reshape(2, NP, D)
    out = _combine_bn_pool_heads(x0, p, row(gamma1), row(beta1), ids3,
                                 fc1_w, row(fc1_b), hw, hb)
    return (out[:, 0:3], out[:, 3:5], out[:, 5:9], out[:, 9:14])


# speedup vs baseline: 12.2016x; 1.0057x over previous
import functools

import jax
import jax.numpy as jnp
from jax import lax
from jax.experimental import pallas as pl
from jax.experimental.pallas import tpu as pltpu
from jax.experimental.pallas import tpu_sc as plsc

N = 10000
D = 128
M = 32
E = 320000
EPAIR = 2 * E
NW = 32
CHUNK = 64
PER_TILE = 20480
EPAD = PER_TILE * NW
NCHUNK = PER_TILE // CHUNK
NP = 10240
ZROWS = NP // 16

R = 5000
G = N // R

_P = None


NBUF = 5
NI = 10
UNROLL = 10


def _sc_edge_scatter(x1, dst2, src2):
    mesh = plsc.VectorSubcoreMesh(core_axis_name="c", subcore_axis_name="s")

    @functools.partial(
        pl.kernel,
        out_type=jax.ShapeDtypeStruct((2 * NP, D), jnp.float32),
        mesh=mesh,
        scratch_types=[
            [pltpu.VMEM((CHUNK,), jnp.int32)] * NI,
            [pltpu.VMEM((CHUNK,), jnp.int32)] * NI,
            [pltpu.VMEM((CHUNK, D), jnp.float32)] * NBUF,
            pltpu.VMEM_SHARED((NP, D), jnp.float32),
            [pltpu.SemaphoreType.DMA] * NI,
            [pltpu.SemaphoreType.DMA] * NBUF,
            [pltpu.SemaphoreType.DMA] * NBUF,
            pltpu.SemaphoreType.DMA,
        ],
    )
    def body(x1_hbm, dst_hbm, src_hbm, out_hbm, dstv, srcv, rbufs, agg,
             isems, rsems, ssems, zsem):
        cid = lax.axis_index("c")
        sid = lax.axis_index("s")
        wid = sid * 2 + cid
        base = wid * NCHUNK

        def idx_issue(slot, h):
            pltpu.async_copy(dst_hbm.at[base + h], dstv[slot], isems[slot])
            pltpu.async_copy(src_hbm.at[base + h], srcv[slot], isems[slot])

        def idx_wait(slot, h):
            pltpu.make_async_copy(dst_hbm.at[base + h], dstv[slot],
                                  isems[slot]).wait()
            pltpu.make_async_copy(src_hbm.at[base + h], srcv[slot],
                                  isems[slot]).wait()

        def gather_issue(slot, islot):
            pltpu.async_copy(x1_hbm.at[srcv[islot]], rbufs[slot], rsems[slot])

        def gather_wait(slot, islot):
            pltpu.make_async_copy(x1_hbm.at[srcv[islot]], rbufs[slot],
                                  rsems[slot]).wait()

        for h in range(NI):
            idx_issue(h, h)

        def zrow(r, carry):
            for c8 in range(8):
                rbufs[0][r, pl.ds(c8 * 16, 16)] = jnp.zeros((16,), jnp.float32)
            return carry

        lax.fori_loop(0, CHUNK, zrow, 0)
        for k in range(ZROWS // CHUNK):
            pltpu.async_copy(rbufs[0],
                             agg.at[pl.ds(sid * ZROWS + k * CHUNK, CHUNK)],
                             zsem)
        for k in range(ZROWS // CHUNK):
            pltpu.make_async_copy(
                rbufs[0], agg.at[pl.ds(sid * ZROWS + k * CHUNK, CHUNK)],
                zsem).wait()

        def scatter_issue(slot, islot):
            pltpu.async_copy(rbufs[slot], agg.at[dstv[islot]], ssems[slot],
                             add=True)

        def scatter_wait(slot, islot):
            pltpu.make_async_copy(rbufs[slot], agg.at[dstv[islot]],
                                  ssems[slot]).wait()

        for h in range(NBUF - 1):
            idx_wait(h, h)
            gather_issue(h, h)
        plsc.subcore_barrier()

        def step(t, carry):
            for u in range(UNROLL):
                g = t * UNROLL + u
                nx = g + NBUF - 1
                ns = (u + NBUF - 1) % NBUF
                ni = (u + NBUF - 1) % NI

                @pl.when(g >= 1)
                def _():
                    scatter_wait(ns, ni)

                @pl.when(nx < NCHUNK)
                def _():
                    idx_wait(ni, nx)
                    gather_issue(ns, ni)

                gather_wait(u % NBUF, u)
                scatter_issue(u % NBUF, u)

                @pl.when((g >= 1) & (g + NI - 1 < NCHUNK))
                def _():
                    idx_issue((u + NI - 1) % NI, g + NI - 1)
            return carry

        lax.fori_loop(0, NCHUNK // UNROLL, step, 0)
        scatter_wait((NCHUNK - 1) % NBUF, (NCHUNK - 1) % NI)
        plsc.subcore_barrier()
        pltpu.sync_copy(
            agg.at[pl.ds(sid * ZROWS, ZROWS)],
            out_hbm.at[pl.ds(cid * NP + sid * ZROWS, ZROWS)],
        )

    return body(x1, dst2, src2)


def _mm2(x, w0, b0, w1, b1):
    def body(x_ref, w0_ref, b0_ref, w1_ref, b1_ref, o0_ref, o1_ref):
        xb = x_ref[...]
        o0_ref[...] = jnp.dot(xb, w0_ref[...], precision=_P,
                              preferred_element_type=jnp.float32) + b0_ref[...]
        o1_ref[...] = jnp.dot(xb, w1_ref[...], precision=_P,
                              preferred_element_type=jnp.float32) + b1_ref[...]

    wspec = pl.BlockSpec((D, D), lambda i: (0, 0))
    bspec = pl.BlockSpec((1, D), lambda i: (0, 0))
    rspec = pl.BlockSpec((R, D), lambda i: (i, 0))
    return pl.pallas_call(
        body,
        grid=(G,),
        in_specs=[rspec, wspec, bspec, wspec, bspec],
        out_specs=[rspec, rspec],
        out_shape=[jax.ShapeDtypeStruct((N, D), jnp.float32)] * 2,
    )(x, w0, b0, w1, b1)


def _bn_stats(y):
    return jnp.concatenate(
        [jnp.sum(y, axis=0, keepdims=True),
         jnp.sum(y * y, axis=0, keepdims=True)], axis=0)


def _bn_relu(y, s_ref, g_ref, be_ref):
    mu = s_ref[0:1, :] * (1.0 / N)
    ex2 = s_ref[1:2, :] * (1.0 / N)
    inv = lax.rsqrt(ex2 - mu * mu + 1e-5)
    return jnp.maximum((y - mu) * (inv * g_ref[...]) + be_ref[...], 0.0)


def _combine_bn_mm2(x0, p, g, be, w0, b0, w1, b1):
    def body(x0_ref, p_ref, g_ref, be_ref, w0_ref, b0_ref, w1_ref, b1_ref,
             o0_ref, o1_ref, y_s, st_s):
        ph = pl.program_id(0)
        i = pl.program_id(1)

        @pl.when(ph == 0)
        def _():
            y = x0_ref[...] + p_ref[0] + p_ref[1]
            y_s[pl.ds(i * R, R), :] = y

            @pl.when(i == 0)
            def _():
                st_s[...] = jnp.zeros_like(st_s)

            st_s[...] += _bn_stats(y)

        @pl.when(ph == 1)
        def _():
            xn = _bn_relu(y_s[pl.ds(i * R, R), :], st_s, g_ref, be_ref)
            o0_ref[...] = jnp.dot(xn, w0_ref[...], precision=_P,
                                  preferred_element_type=jnp.float32) + b0_ref[...]
            o1_ref[...] = jnp.dot(xn, w1_ref[...], precision=_P,
                                  preferred_element_type=jnp.float32) + b1_ref[...]

    wspec = pl.BlockSpec((D, D), lambda ph, i: (0, 0))
    bspec = pl.BlockSpec((1, D), lambda ph, i: (0, 0))
    rspec0 = pl.BlockSpec((R, D), lambda ph, i: (i * (1 - ph), 0))
    rspec1 = pl.BlockSpec((R, D), lambda ph, i: (i * ph, 0))
    return pl.pallas_call(
        body,
        grid=(2, G),
        in_specs=[rspec0, pl.BlockSpec((2, R, D), lambda ph, i: (0, i * (1 - ph), 0)),
                  bspec, bspec, wspec, bspec, wspec, bspec],
        out_specs=[rspec1, rspec1],
        out_shape=[jax.ShapeDtypeStruct((N, D), jnp.float32)] * 2,
        scratch_shapes=[pltpu.VMEM((N, D), jnp.float32),
                        pltpu.VMEM((2, D), jnp.float32)],
    )(x0, p, g, be, w0, b0, w1, b1)


def _combine_bn_pool_heads(x0, p, g, be, ids3, fw, fb, hw, hb):
    def body(x0_ref, p_ref, g_ref, be_ref, ids_ref, fw_ref, fb_ref, hw_ref,
             hb_ref, o_ref, y_s, st_s, seg_s, cnt_s):
        ph = pl.program_id(0)
        i = pl.program_id(1)

        @pl.when(ph == 0)
        def _():
            y = x0_ref[...] + p_ref[0] + p_ref[1]
            y_s[pl.ds(i * R, R), :] = y

            @pl.when(i == 0)
            def _():
                st_s[...] = jnp.zeros_like(st_s)
                seg_s[...] = jnp.zeros_like(seg_s)
                cnt_s[...] = jnp.zeros_like(cnt_s)

            st_s[...] += _bn_stats(y)

        @pl.when(ph == 1)
        def _():
            xn = _bn_relu(y_s[pl.ds(i * R, R), :], st_s, g_ref, be_ref)
            ids = ids_ref[0]
            onehot = (jnp.broadcast_to(ids, (M, R))
                      == lax.broadcasted_iota(jnp.int32, (M, R), 0)
                      ).astype(jnp.float32)
            seg_s[...] += jnp.dot(onehot, xn, precision=_P,
                                  preferred_element_type=jnp.float32)
            cnt_s[...] += jnp.broadcast_to(
                jnp.sum(onehot, axis=1, keepdims=True), (M, D))

            @pl.when(i == G - 1)
            def _():
                mean = seg_s[...] / jnp.maximum(cnt_s[...], 1.0)
                h = jnp.maximum(
                    jnp.dot(mean, fw_ref[...], precision=_P,
                            preferred_element_type=jnp.float32) + fb_ref[...],
                    0.0)
                o_ref[...] = jnp.dot(h, hw_ref[...], precision=_P,
                                     preferred_element_type=jnp.float32
                                     ) + hb_ref[...]

    bspec = pl.BlockSpec((1, D), lambda ph, i: (0, 0))
    wspec = pl.BlockSpec((D, D), lambda ph, i: (0, 0))
    rspec0 = pl.BlockSpec((R, D), lambda ph, i: (i * (1 - ph), 0))
    return pl.pallas_call(
        body,
        grid=(2, G),
        in_specs=[rspec0, pl.BlockSpec((2, R, D), lambda ph, i: (0, i * (1 - ph), 0)),
                  bspec, bspec,
                  pl.BlockSpec((1, 1, R), lambda ph, i: (i * ph, 0, 0)),
                  wspec, bspec, wspec, bspec],
        out_specs=pl.BlockSpec((M, D), lambda ph, i: (0, 0)),
        out_shape=jax.ShapeDtypeStruct((M, D), jnp.float32),
        scratch_shapes=[pltpu.VMEM((N, D), jnp.float32),
                        pltpu.VMEM((2, D), jnp.float32),
                        pltpu.VMEM((M, D), jnp.float32),
                        pltpu.VMEM((M, D), jnp.float32)],
    )(x0, p, g, be, ids3, fw, fb, hw, hb)


def kernel(verts, edges, mesh_idx, W0_0, b0_0, W1_0, b1_0, gamma0, beta0,
           W0_1, b0_1, W1_1, b1_1, gamma1, beta1, fc1_w, fc1_b,
           style_w, style_b, sem_w, sem_b, func_w, func_b, aes_w, aes_b):
    e0 = edges[:, 0].astype(jnp.int32)
    e1 = edges[:, 1].astype(jnp.int32)
    npad = EPAD - EPAIR
    ar = jnp.arange(npad, dtype=jnp.int32)
    dst = jnp.concatenate([e0, e1, N + ar % (NP - N)]
                          ).reshape(EPAD // CHUNK, CHUNK)
    src = jnp.concatenate([e1, e0, ar % N]
                          ).reshape(EPAD // CHUNK, CHUNK)

    row = lambda v: v.reshape(1, D)
    ids3 = mesh_idx.astype(jnp.int32).reshape(G, 1, R)
    hw = jnp.pad(jnp.concatenate([style_w, sem_w, func_w, aes_w], axis=1),
                 ((0, 0), (0, D - 14)))
    hb = jnp.pad(jnp.concatenate([style_b, sem_b, func_b, aes_b]).reshape(1, 14),
                 ((0, 0), (0, D - 14)))

    x0, x1 = _mm2(verts, W0_0, row(b0_0), W1_0, row(b1_0))
    p = _sc_edge_scatter(x1, dst, src).reshape(2, NP, D)
    x0, x1 = _combine_bn_mm2(x0, p, row(gamma0), row(beta0),
                             W0_1, row(b0_1), W1_1, row(b1_1))
    p = _sc_edge_scatter(x1, dst, src).reshape(2, NP, D)
    out = _combine_bn_pool_heads(x0, p, row(gamma1), row(beta1), ids3,
                                 fc1_w, row(fc1_b), hw, hb)
    return (out[:, 0:3], out[:, 3:5], out[:, 5:9], out[:, 9:14])
